# Initial kernel scaffold; baseline (speedup 1.0000x reference)
import jax
import jax.numpy as jnp
from jax.experimental import pallas as pl


def kernel(emb_user, emb_item, ui_vals, iu_vals, d_users, d_items, users, pos_item, neg_item, rows, cols):
    raise NotImplementedError("write your pallas kernel here")


# speedup vs baseline: 1.3274x; 1.3274x over previous
import functools

import jax
import jax.numpy as jnp
from jax.experimental import pallas as pl

N_USERS = 50000
N_ITEMS = 50000
D = 64
LAYERS = 3
ROW_BLK = 2000


def _combine_body(acc_ref, old_ref, a_ref, d_ref, new_ref, s_ref):
    a = a_ref[...]
    new = a * acc_ref[...] + d_ref[...] * old_ref[...]
    new_ref[...] = new
    s_ref[...] = a * new


def _combine(acc, old, a, dvec):
    n = acc.shape[0]
    grid = (n // ROW_BLK,)
    spec_t = pl.BlockSpec((ROW_BLK, D), lambda i: (i, 0))
    spec_v = pl.BlockSpec((ROW_BLK, 1), lambda i: (i, 0))
    return pl.pallas_call(
        _combine_body,
        grid=grid,
        in_specs=[spec_t, spec_t, spec_v, spec_v],
        out_specs=[spec_t, spec_t],
        out_shape=[jax.ShapeDtypeStruct((n, D), jnp.float32)] * 2,
    )(acc, old, a, dvec)


def _prescale_body(old_ref, a_ref, s_ref):
    s_ref[...] = a_ref[...] * old_ref[...]


def _prescale(old, a):
    n = old.shape[0]
    return pl.pallas_call(
        _prescale_body,
        grid=(n // ROW_BLK,),
        in_specs=[pl.BlockSpec((ROW_BLK, D), lambda i: (i, 0)),
                  pl.BlockSpec((ROW_BLK, 1), lambda i: (i, 0))],
        out_specs=pl.BlockSpec((ROW_BLK, D), lambda i: (i, 0)),
        out_shape=jax.ShapeDtypeStruct((n, D), jnp.float32),
    )(old, a)


def kernel(emb_user, emb_item, ui_vals, iu_vals, d_users, d_items,
           users, pos_item, neg_item, rows, cols):
    a = jnp.sqrt(d_users)
    b = jnp.sqrt(d_items)

    gu = [emb_user]
    gi = [emb_item]
    su = _prescale(emb_user, a)
    si = _prescale(emb_item, b)
    for _ in range(LAYERS):
        acc_u = jax.ops.segment_sum(si[cols], rows, num_segments=N_USERS)
        acc_i = jax.ops.segment_sum(su[rows], cols, num_segments=N_ITEMS)
        new_u, su = _combine(acc_u, gu[-1], a, d_users)
        new_i, si = _combine(acc_i, gi[-1], b, d_items)
        gu.append(new_u)
        gi.append(new_i)

    uc = jnp.concatenate(gu, axis=1)
    ic = jnp.concatenate(gi, axis=1)
    return uc[users], ic[pos_item], ic[neg_item]


# speedup vs baseline: 5.7330x; 4.3188x over previous
import jax
import jax.numpy as jnp
from jax import lax
from jax.experimental import pallas as pl
from jax.experimental.pallas import tpu as pltpu
from jax.experimental.pallas import tpu_sc as plsc

N_USERS = 50000
N_ITEMS = 50000
NNZ = 800000
D = 64
LAYERS = 3
BATCH = 4096

_SC_PARAMS = pltpu.CompilerParams(needs_layout_passes=False,
                                  use_tc_tiling_on_sc=False)

NC = 2
NS = 16
LANES = 16

HALF = N_USERS // NC
TRASH = HALF
ACC_ROWS = 25200
COPY_BLK = 200
CHUNK = 128
NCH = 1
GROUP = CHUNK * NCH
NG = NNZ // GROUP
NGS = 392

ROW_BLK = 2000


def _sc_mesh():
    return plsc.VectorSubcoreMesh(core_axis_name="c", subcore_axis_name="s",
                                  num_cores=NC, num_subcores=NS)


def _spmm_body(zrows, si, su, rows, cols, accu, acci,
               dstb, srcb, gbuf, acc,
               sem_i0, sem_i1, sem_g0, sem_g1, sem_s0, sem_s1):
    c = lax.axis_index("c")
    s = lax.axis_index("s")
    lo = c * HALF
    sem_i = (sem_i0, sem_i1)
    sem_g = (sem_g0, sem_g1)
    sem_s = (sem_s0, sem_s1)

    iota16 = lax.iota(jnp.int32, LANES)

    def run_direction(src_table, dst_arr, src_arr, out_hbm):
        @pl.loop(s, ACC_ROWS // COPY_BLK, step=NS)
        def _(ch):
            pltpu.sync_copy(zrows, acc.at[pl.ds(ch * COPY_BLK, COPY_BLK)])
        plsc.subcore_barrier()

        def group_off(g):
            gg = g * NS + s
            return jnp.where(gg < NG, gg, 0) * GROUP, gg < NG

        def issue_idx(g, slot):
            off, _ = group_off(g)
            pltpu.async_copy(dst_arr.at[pl.ds(off, GROUP)], dstb.at[slot],
                             sem_i[slot])
            pltpu.async_copy(src_arr.at[pl.ds(off, GROUP)], srcb.at[slot],
                             sem_i[slot])

        def wait_idx(g, slot):
            off, _ = group_off(g)
            pltpu.make_async_copy(dst_arr.at[pl.ds(off, GROUP)], dstb.at[slot],
                                  sem_i[slot]).wait()
            pltpu.make_async_copy(src_arr.at[pl.ds(off, GROUP)], srcb.at[slot],
                                  sem_i[slot]).wait()

        def issue_gathers(slot):
            for k in range(NCH):
                pltpu.async_copy(
                    src_table.at[srcb.at[slot, pl.ds(k * CHUNK, CHUNK)]],
                    gbuf.at[slot, pl.ds(k * CHUNK, CHUNK)], sem_g[slot])

        def wait_gathers(slot):
            for k in range(NCH):
                pltpu.make_async_copy(
                    src_table.at[srcb.at[slot, pl.ds(k * CHUNK, CHUNK)]],
                    gbuf.at[slot, pl.ds(k * CHUNK, CHUNK)], sem_g[slot]).wait()

        def do_scatters(g, slot):
            _, real = group_off(g)
            for j in range(GROUP // LANES):
                d = dstb[slot, pl.ds(j * LANES, LANES)]
                keep = jnp.logical_and(jnp.logical_and(d >= lo, d < lo + HALF), real)
                loc = jnp.where(keep, d - lo, TRASH + (d & 63))
                pltpu.async_copy(gbuf.at[slot, pl.ds(j * LANES, LANES)],
                                 acc.at[loc], sem_s[slot], add=True)

        def wait_scatters(slot):
            for j in range(GROUP // LANES):
                pltpu.make_async_copy(gbuf.at[slot, pl.ds(j * LANES, LANES)],
                                      acc.at[iota16], sem_s[slot]).wait()

        issue_idx(0, 0)
        wait_idx(0, 0)
        issue_gathers(0)
        issue_idx(1, 1)

        @pl.loop(0, NGS // 2)
        def _(p):
            for b in (0, 1):
                g = p * 2 + b

                @pl.when(g >= 1)
                def _():
                    wait_scatters(1 - b)

                @pl.when(g + 1 < NGS)
                def _():
                    wait_idx(g + 1, 1 - b)
                    issue_gathers(1 - b)

                wait_gathers(b)
                do_scatters(g, b)

                @pl.when(g + 2 < NGS)
                def _():
                    issue_idx(g + 2, b)

        wait_scatters((NGS - 1) % 2)
        plsc.subcore_barrier()

        @pl.loop(s, HALF // COPY_BLK, step=NS)
        def _(ch):
            pltpu.sync_copy(acc.at[pl.ds(ch * COPY_BLK, COPY_BLK)],
                            out_hbm.at[pl.ds(lo + ch * COPY_BLK, COPY_BLK)])
        plsc.subcore_barrier()

    run_direction(si, rows, cols, accu)
    run_direction(su, cols, rows, acci)


def _spmm_pair(zrows, si, su, rows, cols):
    f = pl.kernel(
        _spmm_body,
        out_type=[jax.ShapeDtypeStruct((N_USERS, D), jnp.float32),
                  jax.ShapeDtypeStruct((N_ITEMS, D), jnp.float32)],
        mesh=_sc_mesh(),
        scratch_types=[
            pltpu.VMEM((2, GROUP), jnp.int32),
            pltpu.VMEM((2, GROUP), jnp.int32),
            pltpu.VMEM((2, GROUP, D), jnp.float32),
            pltpu.VMEM_SHARED((ACC_ROWS, D), jnp.float32),
            pltpu.SemaphoreType.DMA,
            pltpu.SemaphoreType.DMA,
            pltpu.SemaphoreType.DMA,
            pltpu.SemaphoreType.DMA,
            pltpu.SemaphoreType.DMA,
            pltpu.SemaphoreType.DMA,
        ],
        compiler_params=_SC_PARAMS,
    )
    return f(zrows, si, su, rows, cols)


def _final_gather_body(tabs, idxs, outs, idxb, gb, sem):
    c = lax.axis_index("c")
    s = lax.axis_index("s")
    w = s * NC + c
    base = w * (BATCH // (NC * NS))
    n = BATCH // (NC * NS)
    for r, idx_arr in enumerate(idxs):
        pltpu.sync_copy(idx_arr.at[pl.ds(base, n)], idxb.at[r])
    for t in range(4 * len(idxs)):
        r = t // 4
        tab = tabs[(0 if r == 0 else 1) * 4 + t % 4]
        pltpu.sync_copy(tab.at[idxb.at[r]], gb)
        pltpu.sync_copy(gb, outs[t].at[pl.ds(base, n)])


def _final_gather(gu, gi, users, pos_item, neg_item):
    def body(u0, u1, u2, u3, i0, i1, i2, i3, users_, pos_, neg_,
             *rest):
        outs = rest[:12]
        idxb, gb, sem = rest[12:]
        _final_gather_body((u0, u1, u2, u3, i0, i1, i2, i3),
                           (users_, pos_, neg_), outs, idxb, gb, sem)

    f = pl.kernel(
        body,
        out_type=[jax.ShapeDtypeStruct((BATCH, D), jnp.float32)] * 12,
        mesh=_sc_mesh(),
        scratch_types=[
            pltpu.VMEM((3, BATCH // (NC * NS)), jnp.int32),
            pltpu.VMEM((BATCH // (NC * NS), D), jnp.float32),
            pltpu.SemaphoreType.DMA,
        ],
        compiler_params=_SC_PARAMS,
    )
    return f(*gu, *gi, users, pos_item, neg_item)


def _combine_body(acc_ref, old_ref, d_ref, new_ref, s_ref):
    a = jnp.sqrt(d_ref[...])
    new = a * acc_ref[...] + d_ref[...] * old_ref[...]
    new_ref[...] = new
    s_ref[...] = a * new


def _combine(acc, old, dvec):
    n = acc.shape[0]
    spec_t = pl.BlockSpec((ROW_BLK, D), lambda i: (i, 0))
    spec_v = pl.BlockSpec((ROW_BLK, 1), lambda i: (i, 0))
    return pl.pallas_call(
        _combine_body,
        grid=(n // ROW_BLK,),
        in_specs=[spec_t, spec_t, spec_v],
        out_specs=[spec_t, spec_t],
        out_shape=[jax.ShapeDtypeStruct((n, D), jnp.float32)] * 2,
    )(acc, old, dvec)


def _prescale_body(old_ref, d_ref, s_ref):
    s_ref[...] = jnp.sqrt(d_ref[...]) * old_ref[...]


def _prescale(old, dvec):
    n = old.shape[0]
    return pl.pallas_call(
        _prescale_body,
        grid=(n // ROW_BLK,),
        in_specs=[pl.BlockSpec((ROW_BLK, D), lambda i: (i, 0)),
                  pl.BlockSpec((ROW_BLK, 1), lambda i: (i, 0))],
        out_specs=pl.BlockSpec((ROW_BLK, D), lambda i: (i, 0)),
        out_shape=jax.ShapeDtypeStruct((n, D), jnp.float32),
    )(old, dvec)


def kernel(emb_user, emb_item, ui_vals, iu_vals, d_users, d_items,
           users, pos_item, neg_item, rows, cols):
    gu = [emb_user]
    gi = [emb_item]
    zrows = jnp.zeros((COPY_BLK, D), jnp.float32)
    su = _prescale(emb_user, d_users)
    si = _prescale(emb_item, d_items)
    for _ in range(LAYERS):
        acc_u, acc_i = _spmm_pair(zrows, si, su, rows, cols)
        new_u, su = _combine(acc_u, gu[-1], d_users)
        new_i, si = _combine(acc_i, gi[-1], d_items)
        gu.append(new_u)
        gi.append(new_i)

    pieces = _final_gather(gu, gi, users, pos_item, neg_item)
    u_emb = jnp.concatenate(pieces[0:4], axis=1)
    pos_emb = jnp.concatenate(pieces[4:8], axis=1)
    neg_emb = jnp.concatenate(pieces[8:12], axis=1)
    return u_emb, pos_emb, neg_emb


# speedup vs baseline: 6.8337x; 1.1920x over previous
import jax
import jax.numpy as jnp
from jax import lax
from jax.experimental import pallas as pl
from jax.experimental.pallas import tpu as pltpu
from jax.experimental.pallas import tpu_sc as plsc

N_USERS = 50000
N_ITEMS = 50000
NNZ = 800000
D = 64
LAYERS = 3
BATCH = 4096

_SC_PARAMS = pltpu.CompilerParams(needs_layout_passes=False,
                                  use_tc_tiling_on_sc=False)

NC = 2
NS = 16
NW = NC * NS
LANES = 16

HALF = N_USERS // NC
TRASH = HALF
ACC_ROWS = 25200
COPY_BLK = 200
CHUNK = 128
NCHT = NNZ // CHUNK
NCHW = 196
SLOT_CAP = NCHW * CHUNK
SENTINEL = 2 * N_USERS

ROW_BLK = 2000


def _sc_mesh():
    return plsc.VectorSubcoreMesh(core_axis_name="c", subcore_axis_name="s",
                                  num_cores=NC, num_subcores=NS)


def _bucket_body(rows, cols, dstU, srcU, dstI, srcI, cnt,
                 ebufd, ebufs, sd0, ss0, sd1, ss1, cntb, sem0, sem1):
    c = lax.axis_index("c")
    s = lax.axis_index("s")
    w = s * NC + c
    nch = 195 + jnp.where(w < 10, 1, 0)
    iota16 = lax.iota(jnp.int32, LANES)
    sems = (sem0, sem1)

    def pass_dir(dst_arr, src_arr, outD, outS, dir_idx, cv_in):
        def issue(j, b):
            off = (j * NW + w) * CHUNK
            pltpu.async_copy(dst_arr.at[pl.ds(off, CHUNK)], ebufd.at[b],
                             sems[b])
            pltpu.async_copy(src_arr.at[pl.ds(off, CHUNK)], ebufs.at[b],
                             sems[b])

        def wait(j, b):
            off = (j * NW + w) * CHUNK
            pltpu.make_async_copy(dst_arr.at[pl.ds(off, CHUNK)],
                                  ebufd.at[b], sems[b]).wait()
            pltpu.make_async_copy(src_arr.at[pl.ds(off, CHUNK)],
                                  ebufs.at[b], sems[b]).wait()

        issue(0, 0)

        def pair(p, carry):
            off0, off1 = carry
            for b in (0, 1):
                j = p * 2 + b
                valid = j < nch

                @pl.when(valid)
                def _():
                    wait(j, b)

                @pl.when(j + 1 < nch)
                def _():
                    issue(j + 1, 1 - b)

                for q in range(CHUNK // LANES):
                    d = ebufd[b, pl.ds(q * LANES, LANES)]
                    sv = ebufs[b, pl.ds(q * LANES, LANES)]
                    m0 = jnp.logical_and(d < HALF, valid)
                    m1 = jnp.logical_and(d >= HALF, valid)
                    i0 = m0.astype(jnp.int32)
                    i1 = m1.astype(jnp.int32)
                    p0 = off0 + plsc.cumsum(i0) - 1
                    p1 = off1 + plsc.cumsum(i1) - 1
                    plsc.store_scatter(sd0, [p0], d, mask=m0)
                    plsc.store_scatter(ss0, [p0], sv, mask=m0)
                    plsc.store_scatter(sd1, [p1], d, mask=m1)
                    plsc.store_scatter(ss1, [p1], sv, mask=m1)
                    off0 = off0 + jnp.sum(i0)
                    off1 = off1 + jnp.sum(i1)
            return off0, off1

        n0, n1 = lax.fori_loop(0, NCHW // 2, pair,
                               (jnp.int32(0), jnp.int32(0)))

        def finish(n, sdst, ssrc, outDref, outSref):
            pend = jnp.minimum((n + CHUNK - 1) & (-CHUNK), SLOT_CAP)
            sent = jnp.full((LANES,), SENTINEL, jnp.int32)
            zero = jnp.zeros((LANES,), jnp.int32)
            for t in range(CHUNK // LANES):
                pos = n + t * LANES + iota16
                m = pos < pend
                plsc.store_scatter(sdst, [pos], sent, mask=m)
                plsc.store_scatter(ssrc, [pos], zero, mask=m)
            pltpu.sync_copy(sdst, outDref)
            pltpu.sync_copy(ssrc, outSref)
            return pend // CHUNK

        nch0 = finish(n0, sd0, ss0, outD.at[0, w], outS.at[0, w])
        nch1 = finish(n1, sd1, ss1, outD.at[1, w], outS.at[1, w])
        cv = jnp.where(iota16 == 2 * dir_idx, nch0,
                       jnp.where(iota16 == 2 * dir_idx + 1, nch1, cv_in))
        return cv

    cv = jnp.zeros((LANES,), jnp.int32)
    cv = pass_dir(rows, cols, dstU, srcU, 0, cv)
    cv = pass_dir(cols, rows, dstI, srcI, 1, cv)
    cntb[...] = cv
    pltpu.sync_copy(cntb, cnt.at[w])


def _bucket(rows, cols):
    slot = jax.ShapeDtypeStruct((2, NW, SLOT_CAP), jnp.int32)
    f = pl.kernel(
        _bucket_body,
        out_type=[slot, slot, slot, slot,
                  jax.ShapeDtypeStruct((NW, LANES), jnp.int32)],
        mesh=_sc_mesh(),
        scratch_types=[
            pltpu.VMEM((2, CHUNK), jnp.int32),
            pltpu.VMEM((2, CHUNK), jnp.int32),
            pltpu.VMEM((SLOT_CAP,), jnp.int32),
            pltpu.VMEM((SLOT_CAP,), jnp.int32),
            pltpu.VMEM((SLOT_CAP,), jnp.int32),
            pltpu.VMEM((SLOT_CAP,), jnp.int32),
            pltpu.VMEM((LANES,), jnp.int32),
            pltpu.SemaphoreType.DMA,
            pltpu.SemaphoreType.DMA,
        ],
        compiler_params=_SC_PARAMS,
    )
    return f(rows, cols)


def _spmm_body(zrows, si, su, dstU, srcU, dstI, srcI, cnt, accu, acci,
               dstb, srcb, gbuf, cbuf, acc,
               sem_i0, sem_i1, sem_g0, sem_g1, sem_s0, sem_s1):
    c = lax.axis_index("c")
    s = lax.axis_index("s")
    lo = c * HALF
    sem_i = (sem_i0, sem_i1)
    sem_g = (sem_g0, sem_g1)
    sem_s = (sem_s0, sem_s1)

    iota16 = lax.iota(jnp.int32, LANES)

    def get_count(slot, col):
        pltpu.sync_copy(cnt.at[slot], cbuf)
        v = cbuf[...]
        return jnp.sum(jnp.where(iota16 == col, v, 0))

    def run_direction(src_table, dArr, sArr, dir_idx, out_hbm):
        @pl.loop(s, ACC_ROWS // COPY_BLK, step=NS)
        def _(ch):
            pltpu.sync_copy(zrows, acc.at[pl.ds(ch * COPY_BLK, COPY_BLK)])
        plsc.subcore_barrier()

        col = 2 * dir_idx + c
        nA = get_count(s, col)
        nB = get_count(s + NS, col)
        T = nA + nB

        def chunk_src(t, arr):
            inA = t < nA
            slot = jnp.where(inA, s, s + NS)
            chv = jnp.where(inA, t, t - nA)
            return arr.at[c, slot, pl.ds(chv * CHUNK, CHUNK)]

        def issue_idx(t, b):
            pltpu.async_copy(chunk_src(t, dArr), dstb.at[b], sem_i[b])
            pltpu.async_copy(chunk_src(t, sArr), srcb.at[b], sem_i[b])

        def wait_idx(t, b):
            pltpu.make_async_copy(chunk_src(t, dArr), dstb.at[b],
                                  sem_i[b]).wait()
            pltpu.make_async_copy(chunk_src(t, sArr), srcb.at[b],
                                  sem_i[b]).wait()

        def issue_gather(b):
            pltpu.async_copy(src_table.at[srcb.at[b]], gbuf.at[b], sem_g[b])

        def wait_gather(b):
            pltpu.make_async_copy(src_table.at[srcb.at[b]], gbuf.at[b],
                                  sem_g[b]).wait()

        def do_scatters(b):
            for j in range(CHUNK // LANES):
                d = dstb[b, pl.ds(j * LANES, LANES)]
                keep = jnp.logical_and(d >= lo, d < lo + HALF)
                loc = jnp.where(keep, d - lo, TRASH + (d & 63))
                pltpu.async_copy(gbuf.at[b, pl.ds(j * LANES, LANES)],
                                 acc.at[loc], sem_s[b], add=True)

        def wait_scatters(b):
            for j in range(CHUNK // LANES):
                pltpu.make_async_copy(gbuf.at[b, pl.ds(j * LANES, LANES)],
                                      acc.at[iota16], sem_s[b]).wait()

        @pl.when(T > 0)
        def _():
            issue_idx(0, 0)
            wait_idx(0, 0)
            issue_gather(0)

        @pl.when(T > 1)
        def _():
            issue_idx(1, 1)

        @pl.loop(0, (T + 1) // 2)
        def _(p):
            for b in (0, 1):
                t = p * 2 + b

                @pl.when(jnp.logical_and(t >= 1, t <= T - 1))
                def _():
                    wait_scatters(1 - b)

                @pl.when(t + 1 < T)
                def _():
                    wait_idx(t + 1, 1 - b)
                    issue_gather(1 - b)

                @pl.when(t < T)
                def _():
                    wait_gather(b)
                    do_scatters(b)

                @pl.when(t + 2 < T)
                def _():
                    issue_idx(t + 2, b)

        @pl.when(T > 0)
        def _():
            @pl.when((T - 1) % 2 == 0)
            def _():
                wait_scatters(0)

            @pl.when((T - 1) % 2 == 1)
            def _():
                wait_scatters(1)

        plsc.subcore_barrier()

        @pl.loop(s, HALF // COPY_BLK, step=NS)
        def _(ch):
            pltpu.sync_copy(acc.at[pl.ds(ch * COPY_BLK, COPY_BLK)],
                            out_hbm.at[pl.ds(lo + ch * COPY_BLK, COPY_BLK)])
        plsc.subcore_barrier()

    run_direction(si, dstU, srcU, 0, accu)
    run_direction(su, dstI, srcI, 1, acci)


def _spmm_pair(zrows, si, su, dstU, srcU, dstI, srcI, cnt):
    f = pl.kernel(
        _spmm_body,
        out_type=[jax.ShapeDtypeStruct((N_USERS, D), jnp.float32),
                  jax.ShapeDtypeStruct((N_ITEMS, D), jnp.float32)],
        mesh=_sc_mesh(),
        scratch_types=[
            pltpu.VMEM((2, CHUNK), jnp.int32),
            pltpu.VMEM((2, CHUNK), jnp.int32),
            pltpu.VMEM((2, CHUNK, D), jnp.float32),
            pltpu.VMEM((LANES,), jnp.int32),
            pltpu.VMEM_SHARED((ACC_ROWS, D), jnp.float32),
            pltpu.SemaphoreType.DMA,
            pltpu.SemaphoreType.DMA,
            pltpu.SemaphoreType.DMA,
            pltpu.SemaphoreType.DMA,
            pltpu.SemaphoreType.DMA,
            pltpu.SemaphoreType.DMA,
        ],
        compiler_params=_SC_PARAMS,
    )
    return f(zrows, si, su, dstU, srcU, dstI, srcI, cnt)


def _final_gather_body(tabs, idxs, outs, idxb, gb, sem):
    c = lax.axis_index("c")
    s = lax.axis_index("s")
    w = s * NC + c
    n = BATCH // NW
    base = w * n
    for r, idx_arr in enumerate(idxs):
        pltpu.sync_copy(idx_arr.at[pl.ds(base, n)], idxb.at[r])
    for t in range(4 * len(idxs)):
        r = t // 4
        tab = tabs[(0 if r == 0 else 1) * 4 + t % 4]
        pltpu.sync_copy(tab.at[idxb.at[r]], gb)
        pltpu.sync_copy(gb, outs[t].at[pl.ds(base, n)])


def _final_gather(gu, gi, users, pos_item, neg_item):
    def body(u0, u1, u2, u3, i0, i1, i2, i3, users_, pos_, neg_, *rest):
        outs = rest[:12]
        idxb, gb, sem = rest[12:]
        _final_gather_body((u0, u1, u2, u3, i0, i1, i2, i3),
                           (users_, pos_, neg_), outs, idxb, gb, sem)

    f = pl.kernel(
        body,
        out_type=[jax.ShapeDtypeStruct((BATCH, D), jnp.float32)] * 12,
        mesh=_sc_mesh(),
        scratch_types=[
            pltpu.VMEM((3, BATCH // NW), jnp.int32),
            pltpu.VMEM((BATCH // NW, D), jnp.float32),
            pltpu.SemaphoreType.DMA,
        ],
        compiler_params=_SC_PARAMS,
    )
    return f(*gu, *gi, users, pos_item, neg_item)


def _combine_body(acc_ref, old_ref, d_ref, new_ref, s_ref):
    a = jnp.sqrt(d_ref[...])
    new = a * acc_ref[...] + d_ref[...] * old_ref[...]
    new_ref[...] = new
    s_ref[...] = a * new


def _combine(acc, old, dvec):
    n = acc.shape[0]
    spec_t = pl.BlockSpec((ROW_BLK, D), lambda i: (i, 0))
    spec_v = pl.BlockSpec((ROW_BLK, 1), lambda i: (i, 0))
    return pl.pallas_call(
        _combine_body,
        grid=(n // ROW_BLK,),
        in_specs=[spec_t, spec_t, spec_v],
        out_specs=[spec_t, spec_t],
        out_shape=[jax.ShapeDtypeStruct((n, D), jnp.float32)] * 2,
    )(acc, old, dvec)


def _prescale_body(old_ref, d_ref, s_ref):
    s_ref[...] = jnp.sqrt(d_ref[...]) * old_ref[...]


def _prescale(old, dvec):
    n = old.shape[0]
    return pl.pallas_call(
        _prescale_body,
        grid=(n // ROW_BLK,),
        in_specs=[pl.BlockSpec((ROW_BLK, D), lambda i: (i, 0)),
                  pl.BlockSpec((ROW_BLK, 1), lambda i: (i, 0))],
        out_specs=pl.BlockSpec((ROW_BLK, D), lambda i: (i, 0)),
        out_shape=jax.ShapeDtypeStruct((n, D), jnp.float32),
    )(old, dvec)


def kernel(emb_user, emb_item, ui_vals, iu_vals, d_users, d_items,
           users, pos_item, neg_item, rows, cols):
    gu = [emb_user]
    gi = [emb_item]
    zrows = jnp.zeros((COPY_BLK, D), jnp.float32)
    dstU, srcU, dstI, srcI, cnt = _bucket(rows, cols)
    su = _prescale(emb_user, d_users)
    si = _prescale(emb_item, d_items)
    for _ in range(LAYERS):
        acc_u, acc_i = _spmm_pair(zrows, si, su, dstU, srcU, dstI, srcI, cnt)
        new_u, su = _combine(acc_u, gu[-1], d_users)
        new_i, si = _combine(acc_i, gi[-1], d_items)
        gu.append(new_u)
        gi.append(new_i)

    pieces = _final_gather(gu, gi, users, pos_item, neg_item)
    u_emb = jnp.concatenate(pieces[0:4], axis=1)
    pos_emb = jnp.concatenate(pieces[4:8], axis=1)
    neg_emb = jnp.concatenate(pieces[8:12], axis=1)
    return u_emb, pos_emb, neg_emb


# speedup vs baseline: 7.0549x; 1.0324x over previous
import jax
import jax.numpy as jnp
from jax import lax
from jax.experimental import pallas as pl
from jax.experimental.pallas import tpu as pltpu
from jax.experimental.pallas import tpu_sc as plsc

N_USERS = 50000
N_ITEMS = 50000
NNZ = 800000
D = 64
LAYERS = 3
BATCH = 4096

_SC_PARAMS = pltpu.CompilerParams(needs_layout_passes=False,
                                  use_tc_tiling_on_sc=False)

NC = 2
NS = 16
NW = NC * NS
LANES = 16

HALF = N_USERS // NC
TRASH = HALF
ACC_ROWS = 25200
COPY_BLK = 200
ZBLK = ACC_ROWS // NS
CHUNK = 128
NCHT = NNZ // CHUNK
NCHW = 196
SLOT_CAP = NCHW * CHUNK
SENTINEL = 2 * N_USERS

ROW_BLK = 2000


def _sc_mesh():
    return plsc.VectorSubcoreMesh(core_axis_name="c", subcore_axis_name="s",
                                  num_cores=NC, num_subcores=NS)


def _bucket_body(rows, cols, dstU, srcU, dstI, srcI, cnt,
                 ebufd, ebufs, sd0, ss0, sd1, ss1, cntb, sem0, sem1):
    c = lax.axis_index("c")
    s = lax.axis_index("s")
    w = s * NC + c
    nch = 195 + jnp.where(w < 10, 1, 0)
    iota16 = lax.iota(jnp.int32, LANES)
    sems = (sem0, sem1)

    def pass_dir(dst_arr, src_arr, outD, outS, dir_idx, cv_in):
        def issue(j, b):
            off = (j * NW + w) * CHUNK
            pltpu.async_copy(dst_arr.at[pl.ds(off, CHUNK)], ebufd.at[b],
                             sems[b])
            pltpu.async_copy(src_arr.at[pl.ds(off, CHUNK)], ebufs.at[b],
                             sems[b])

        def wait(j, b):
            off = (j * NW + w) * CHUNK
            pltpu.make_async_copy(dst_arr.at[pl.ds(off, CHUNK)],
                                  ebufd.at[b], sems[b]).wait()
            pltpu.make_async_copy(src_arr.at[pl.ds(off, CHUNK)],
                                  ebufs.at[b], sems[b]).wait()

        issue(0, 0)

        def pair(p, carry):
            off0, off1 = carry
            for b in (0, 1):
                j = p * 2 + b
                valid = j < nch

                @pl.when(valid)
                def _():
                    wait(j, b)

                @pl.when(j + 1 < nch)
                def _():
                    issue(j + 1, 1 - b)

                for q in range(CHUNK // LANES):
                    d = ebufd[b, pl.ds(q * LANES, LANES)]
                    sv = ebufs[b, pl.ds(q * LANES, LANES)]
                    m0 = jnp.logical_and(d < HALF, valid)
                    m1 = jnp.logical_and(d >= HALF, valid)
                    i0 = m0.astype(jnp.int32)
                    i1 = m1.astype(jnp.int32)
                    p0 = off0 + plsc.cumsum(i0) - 1
                    p1 = off1 + plsc.cumsum(i1) - 1
                    plsc.store_scatter(sd0, [p0], d, mask=m0)
                    plsc.store_scatter(ss0, [p0], sv, mask=m0)
                    plsc.store_scatter(sd1, [p1], d, mask=m1)
                    plsc.store_scatter(ss1, [p1], sv, mask=m1)
                    off0 = off0 + jnp.sum(i0)
                    off1 = off1 + jnp.sum(i1)
            return off0, off1

        n0, n1 = lax.fori_loop(0, NCHW // 2, pair,
                               (jnp.int32(0), jnp.int32(0)))

        def finish(n, sdst, ssrc, outDref, outSref):
            pend = jnp.minimum((n + CHUNK - 1) & (-CHUNK), SLOT_CAP)
            sent = jnp.full((LANES,), SENTINEL, jnp.int32)
            zero = jnp.zeros((LANES,), jnp.int32)
            for t in range(CHUNK // LANES):
                pos = n + t * LANES + iota16
                m = pos < pend
                plsc.store_scatter(sdst, [pos], sent, mask=m)
                plsc.store_scatter(ssrc, [pos], zero, mask=m)
            pltpu.sync_copy(sdst, outDref)
            pltpu.sync_copy(ssrc, outSref)
            return pend // CHUNK

        nch0 = finish(n0, sd0, ss0, outD.at[0, w], outS.at[0, w])
        nch1 = finish(n1, sd1, ss1, outD.at[1, w], outS.at[1, w])
        cv = jnp.where(iota16 == 2 * dir_idx, nch0,
                       jnp.where(iota16 == 2 * dir_idx + 1, nch1, cv_in))
        return cv

    cv = jnp.zeros((LANES,), jnp.int32)
    cv = pass_dir(rows, cols, dstU, srcU, 0, cv)
    cv = pass_dir(cols, rows, dstI, srcI, 1, cv)
    cntb[...] = cv
    pltpu.sync_copy(cntb, cnt.at[w])


def _bucket(rows, cols):
    slot = jax.ShapeDtypeStruct((2, NW, SLOT_CAP), jnp.int32)
    f = pl.kernel(
        _bucket_body,
        out_type=[slot, slot, slot, slot,
                  jax.ShapeDtypeStruct((NW, LANES), jnp.int32)],
        mesh=_sc_mesh(),
        scratch_types=[
            pltpu.VMEM((2, CHUNK), jnp.int32),
            pltpu.VMEM((2, CHUNK), jnp.int32),
            pltpu.VMEM((SLOT_CAP,), jnp.int32),
            pltpu.VMEM((SLOT_CAP,), jnp.int32),
            pltpu.VMEM((SLOT_CAP,), jnp.int32),
            pltpu.VMEM((SLOT_CAP,), jnp.int32),
            pltpu.VMEM((LANES,), jnp.int32),
            pltpu.SemaphoreType.DMA,
            pltpu.SemaphoreType.DMA,
        ],
        compiler_params=_SC_PARAMS,
    )
    return f(rows, cols)


def _spmm_body(zrows, si, su, dstU, srcU, dstI, srcI, cnt, accu, acci,
               dstb, srcb, gbuf, cbuf, loc0, loc1, acc,
               sem_i0, sem_i1, sem_g0, sem_g1, sem_s0, sem_s1):
    c = lax.axis_index("c")
    s = lax.axis_index("s")
    lo = c * HALF
    sem_i = (sem_i0, sem_i1)
    sem_g = (sem_g0, sem_g1)
    sem_s = (sem_s0, sem_s1)

    iota16 = lax.iota(jnp.int32, LANES)

    def get_count(slot, col):
        pltpu.sync_copy(cnt.at[slot], cbuf)
        v = cbuf[...]
        return jnp.sum(jnp.where(iota16 == col, v, 0))

    locs = (loc0, loc1)

    def run_direction(src_table, dArr, sArr, dir_idx, out_hbm):
        pltpu.sync_copy(zrows, acc.at[pl.ds(s * ZBLK, ZBLK)])
        plsc.subcore_barrier()

        col = 2 * dir_idx + c
        nA = get_count(s, col)
        nB = get_count(s + NS, col)
        T = nA + nB

        def chunk_src(t, arr):
            inA = t < nA
            slot = jnp.where(inA, s, s + NS)
            chv = jnp.where(inA, t, t - nA)
            return arr.at[c, slot, pl.ds(chv * CHUNK, CHUNK)]

        def issue_idx(t, b):
            pltpu.async_copy(chunk_src(t, dArr), dstb.at[b], sem_i[b])
            pltpu.async_copy(chunk_src(t, sArr), srcb.at[b], sem_i[b])

        def wait_idx(t, b):
            pltpu.make_async_copy(chunk_src(t, dArr), dstb.at[b],
                                  sem_i[b]).wait()
            pltpu.make_async_copy(chunk_src(t, sArr), srcb.at[b],
                                  sem_i[b]).wait()

        def issue_gather(b):
            pltpu.async_copy(src_table.at[srcb.at[b]], gbuf.at[b], sem_g[b])

        def wait_gather(b):
            pltpu.make_async_copy(src_table.at[srcb.at[b]], gbuf.at[b],
                                  sem_g[b]).wait()

        def do_scatters(b):
            lref = locs[b]
            for j in range(CHUNK // LANES):
                d = dstb[b, pl.ds(j * LANES, LANES)]
                keep = jnp.logical_and(d >= lo, d < lo + HALF)
                loc = jnp.where(keep, d - lo, TRASH + (d & 63))
                plsc.store_scatter(lref, [j * LANES + iota16], loc)
            pltpu.async_copy(gbuf.at[b], acc.at[lref], sem_s[b], add=True)

        def wait_scatters(b):
            pltpu.make_async_copy(gbuf.at[b], acc.at[locs[b]],
                                  sem_s[b]).wait()

        @pl.when(T > 0)
        def _():
            issue_idx(0, 0)
            wait_idx(0, 0)
            issue_gather(0)

        @pl.when(T > 1)
        def _():
            issue_idx(1, 1)

        @pl.loop(0, (T + 1) // 2)
        def _(p):
            for b in (0, 1):
                t = p * 2 + b

                @pl.when(jnp.logical_and(t >= 1, t <= T - 1))
                def _():
                    wait_scatters(1 - b)

                @pl.when(t + 1 < T)
                def _():
                    wait_idx(t + 1, 1 - b)
                    issue_gather(1 - b)

                @pl.when(t < T)
                def _():
                    wait_gather(b)
                    do_scatters(b)

                @pl.when(t + 2 < T)
                def _():
                    issue_idx(t + 2, b)

        @pl.when(T > 0)
        def _():
            @pl.when((T - 1) % 2 == 0)
            def _():
                wait_scatters(0)

            @pl.when((T - 1) % 2 == 1)
            def _():
                wait_scatters(1)

        plsc.subcore_barrier()

        @pl.loop(s, HALF // COPY_BLK, step=NS)
        def _(ch):
            pltpu.sync_copy(acc.at[pl.ds(ch * COPY_BLK, COPY_BLK)],
                            out_hbm.at[pl.ds(lo + ch * COPY_BLK, COPY_BLK)])
        plsc.subcore_barrier()

    run_direction(si, dstU, srcU, 0, accu)
    run_direction(su, dstI, srcI, 1, acci)


def _spmm_pair(zrows, si, su, dstU, srcU, dstI, srcI, cnt):
    f = pl.kernel(
        _spmm_body,
        out_type=[jax.ShapeDtypeStruct((N_USERS, D), jnp.float32),
                  jax.ShapeDtypeStruct((N_ITEMS, D), jnp.float32)],
        mesh=_sc_mesh(),
        scratch_types=[
            pltpu.VMEM((2, CHUNK), jnp.int32),
            pltpu.VMEM((2, CHUNK), jnp.int32),
            pltpu.VMEM((2, CHUNK, D), jnp.float32),
            pltpu.VMEM((LANES,), jnp.int32),
            pltpu.VMEM((CHUNK,), jnp.int32),
            pltpu.VMEM((CHUNK,), jnp.int32),
            pltpu.VMEM_SHARED((ACC_ROWS, D), jnp.float32),
            pltpu.SemaphoreType.DMA,
            pltpu.SemaphoreType.DMA,
            pltpu.SemaphoreType.DMA,
            pltpu.SemaphoreType.DMA,
            pltpu.SemaphoreType.DMA,
            pltpu.SemaphoreType.DMA,
        ],
        compiler_params=_SC_PARAMS,
    )
    return f(zrows, si, su, dstU, srcU, dstI, srcI, cnt)


def _final_gather_body(tabs, idxs, outs, idxb, gb, sem):
    c = lax.axis_index("c")
    s = lax.axis_index("s")
    w = s * NC + c
    n = BATCH // NW
    base = w * n
    for r, idx_arr in enumerate(idxs):
        pltpu.sync_copy(idx_arr.at[pl.ds(base, n)], idxb.at[r])
    for t in range(4 * len(idxs)):
        r = t // 4
        tab = tabs[(0 if r == 0 else 1) * 4 + t % 4]
        pltpu.sync_copy(tab.at[idxb.at[r]], gb)
        pltpu.sync_copy(gb, outs[t].at[pl.ds(base, n)])


def _final_gather(gu, gi, users, pos_item, neg_item):
    def body(u0, u1, u2, u3, i0, i1, i2, i3, users_, pos_, neg_, *rest):
        outs = rest[:12]
        idxb, gb, sem = rest[12:]
        _final_gather_body((u0, u1, u2, u3, i0, i1, i2, i3),
                           (users_, pos_, neg_), outs, idxb, gb, sem)

    f = pl.kernel(
        body,
        out_type=[jax.ShapeDtypeStruct((BATCH, D), jnp.float32)] * 12,
        mesh=_sc_mesh(),
        scratch_types=[
            pltpu.VMEM((3, BATCH // NW), jnp.int32),
            pltpu.VMEM((BATCH // NW, D), jnp.float32),
            pltpu.SemaphoreType.DMA,
        ],
        compiler_params=_SC_PARAMS,
    )
    return f(*gu, *gi, users, pos_item, neg_item)


def _combine_body(acc_ref, old_ref, d_ref, new_ref, s_ref):
    a = jnp.sqrt(d_ref[...])
    new = a * acc_ref[...] + d_ref[...] * old_ref[...]
    new_ref[...] = new
    s_ref[...] = a * new


def _combine(acc, old, dvec):
    n = acc.shape[0]
    spec_t = pl.BlockSpec((ROW_BLK, D), lambda i: (i, 0))
    spec_v = pl.BlockSpec((ROW_BLK, 1), lambda i: (i, 0))
    return pl.pallas_call(
        _combine_body,
        grid=(n // ROW_BLK,),
        in_specs=[spec_t, spec_t, spec_v],
        out_specs=[spec_t, spec_t],
        out_shape=[jax.ShapeDtypeStruct((n, D), jnp.float32)] * 2,
    )(acc, old, dvec)


def _prescale_body(old_ref, d_ref, s_ref):
    s_ref[...] = jnp.sqrt(d_ref[...]) * old_ref[...]


def _prescale(old, dvec):
    n = old.shape[0]
    return pl.pallas_call(
        _prescale_body,
        grid=(n // ROW_BLK,),
        in_specs=[pl.BlockSpec((ROW_BLK, D), lambda i: (i, 0)),
                  pl.BlockSpec((ROW_BLK, 1), lambda i: (i, 0))],
        out_specs=pl.BlockSpec((ROW_BLK, D), lambda i: (i, 0)),
        out_shape=jax.ShapeDtypeStruct((n, D), jnp.float32),
    )(old, dvec)


def kernel(emb_user, emb_item, ui_vals, iu_vals, d_users, d_items,
           users, pos_item, neg_item, rows, cols):
    gu = [emb_user]
    gi = [emb_item]
    zrows = jnp.zeros((ZBLK, D), jnp.float32)
    dstU, srcU, dstI, srcI, cnt = _bucket(rows, cols)
    su = _prescale(emb_user, d_users)
    si = _prescale(emb_item, d_items)
    for _ in range(LAYERS):
        acc_u, acc_i = _spmm_pair(zrows, si, su, dstU, srcU, dstI, srcI, cnt)
        new_u, su = _combine(acc_u, gu[-1], d_users)
        new_i, si = _combine(acc_i, gi[-1], d_items)
        gu.append(new_u)
        gi.append(new_i)

    pieces = _final_gather(gu, gi, users, pos_item, neg_item)
    u_emb = jnp.concatenate(pieces[0:4], axis=1)
    pos_emb = jnp.concatenate(pieces[4:8], axis=1)
    neg_emb = jnp.concatenate(pieces[8:12], axis=1)
    return u_emb, pos_emb, neg_emb


# speedup vs baseline: 7.1365x; 1.0116x over previous
import jax
import jax.numpy as jnp
from jax import lax
from jax.experimental import pallas as pl
from jax.experimental.pallas import tpu as pltpu
from jax.experimental.pallas import tpu_sc as plsc

N_USERS = 50000
N_ITEMS = 50000
NNZ = 800000
D = 64
LAYERS = 3
BATCH = 4096

_SC_PARAMS = pltpu.CompilerParams(needs_layout_passes=False,
                                  use_tc_tiling_on_sc=False)

NC = 2
NS = 16
NW = NC * NS
LANES = 16

HALF = N_USERS // NC
TRASH = HALF
ACC_ROWS = 25200
COPY_BLK = 200
ZBLK = ACC_ROWS // NS
CHUNK = 128
NCHT = NNZ // CHUNK
NCHW = 196
SLOT_CAP = NCHW * CHUNK
SENTINEL = 2 * N_USERS

ROW_BLK = 2000


def _sc_mesh():
    return plsc.VectorSubcoreMesh(core_axis_name="c", subcore_axis_name="s",
                                  num_cores=NC, num_subcores=NS)


def _bucket_body(rows, cols, dstU, srcU, dstI, srcI, cnt,
                 ebufd, ebufs, sd0, ss0, sd1, ss1, cntb, sem0, sem1):
    c = lax.axis_index("c")
    s = lax.axis_index("s")
    w = s * NC + c
    nch = 195 + jnp.where(w < 10, 1, 0)
    iota16 = lax.iota(jnp.int32, LANES)
    sems = (sem0, sem1)

    def pass_dir(dst_arr, src_arr, outD, outS, dir_idx, cv_in):
        def issue(j, b):
            off = (j * NW + w) * CHUNK
            pltpu.async_copy(dst_arr.at[pl.ds(off, CHUNK)], ebufd.at[b],
                             sems[b])
            pltpu.async_copy(src_arr.at[pl.ds(off, CHUNK)], ebufs.at[b],
                             sems[b])

        def wait(j, b):
            off = (j * NW + w) * CHUNK
            pltpu.make_async_copy(dst_arr.at[pl.ds(off, CHUNK)],
                                  ebufd.at[b], sems[b]).wait()
            pltpu.make_async_copy(src_arr.at[pl.ds(off, CHUNK)],
                                  ebufs.at[b], sems[b]).wait()

        issue(0, 0)

        def pair(p, carry):
            off0, off1 = carry
            for b in (0, 1):
                j = p * 2 + b
                valid = j < nch

                @pl.when(valid)
                def _():
                    wait(j, b)

                @pl.when(j + 1 < nch)
                def _():
                    issue(j + 1, 1 - b)

                for q in range(CHUNK // LANES):
                    d = ebufd[b, pl.ds(q * LANES, LANES)]
                    sv = ebufs[b, pl.ds(q * LANES, LANES)]
                    m0 = jnp.logical_and(d < HALF, valid)
                    m1 = jnp.logical_and(d >= HALF, valid)
                    i0 = m0.astype(jnp.int32)
                    cs0 = plsc.cumsum(i0)
                    p0 = off0 + cs0 - 1
                    p1 = off1 + iota16 - cs0
                    plsc.store_scatter(sd0, [p0], d, mask=m0)
                    plsc.store_scatter(ss0, [p0], sv, mask=m0)
                    plsc.store_scatter(sd1, [p1], d, mask=m1)
                    plsc.store_scatter(ss1, [p1], sv, mask=m1)
                    s0 = jnp.sum(i0)
                    off0 = off0 + s0
                    off1 = off1 + jnp.where(valid, LANES - s0, 0)
            return off0, off1

        n0, n1 = lax.fori_loop(0, NCHW // 2, pair,
                               (jnp.int32(0), jnp.int32(0)))

        def finish(n, sdst, ssrc, outDref, outSref):
            pend = jnp.minimum((n + CHUNK - 1) & (-CHUNK), SLOT_CAP)
            sent = jnp.full((LANES,), SENTINEL, jnp.int32)
            zero = jnp.zeros((LANES,), jnp.int32)
            for t in range(CHUNK // LANES):
                pos = n + t * LANES + iota16
                m = pos < pend
                plsc.store_scatter(sdst, [pos], sent, mask=m)
                plsc.store_scatter(ssrc, [pos], zero, mask=m)
            pltpu.sync_copy(sdst, outDref)
            pltpu.sync_copy(ssrc, outSref)
            return pend // CHUNK

        nch0 = finish(n0, sd0, ss0, outD.at[0, w], outS.at[0, w])
        nch1 = finish(n1, sd1, ss1, outD.at[1, w], outS.at[1, w])
        cv = jnp.where(iota16 == 2 * dir_idx, nch0,
                       jnp.where(iota16 == 2 * dir_idx + 1, nch1, cv_in))
        return cv

    cv = jnp.zeros((LANES,), jnp.int32)
    cv = pass_dir(rows, cols, dstU, srcU, 0, cv)
    cv = pass_dir(cols, rows, dstI, srcI, 1, cv)
    cntb[...] = cv
    pltpu.sync_copy(cntb, cnt.at[w])


def _bucket(rows, cols):
    slot = jax.ShapeDtypeStruct((2, NW, SLOT_CAP), jnp.int32)
    f = pl.kernel(
        _bucket_body,
        out_type=[slot, slot, slot, slot,
                  jax.ShapeDtypeStruct((NW, LANES), jnp.int32)],
        mesh=_sc_mesh(),
        scratch_types=[
            pltpu.VMEM((2, CHUNK), jnp.int32),
            pltpu.VMEM((2, CHUNK), jnp.int32),
            pltpu.VMEM((SLOT_CAP,), jnp.int32),
            pltpu.VMEM((SLOT_CAP,), jnp.int32),
            pltpu.VMEM((SLOT_CAP,), jnp.int32),
            pltpu.VMEM((SLOT_CAP,), jnp.int32),
            pltpu.VMEM((LANES,), jnp.int32),
            pltpu.SemaphoreType.DMA,
            pltpu.SemaphoreType.DMA,
        ],
        compiler_params=_SC_PARAMS,
    )
    return f(rows, cols)


def _spmm_body(zrows, si, su, dstU, srcU, dstI, srcI, cnt, accu, acci,
               dstb, srcb, gbuf, cbuf, loc0, loc1, acc,
               sem_i0, sem_i1, sem_g0, sem_g1, sem_s0, sem_s1):
    c = lax.axis_index("c")
    s = lax.axis_index("s")
    lo = c * HALF
    sem_i = (sem_i0, sem_i1)
    sem_g = (sem_g0, sem_g1)
    sem_s = (sem_s0, sem_s1)

    iota16 = lax.iota(jnp.int32, LANES)

    def get_count(slot, col):
        pltpu.sync_copy(cnt.at[slot], cbuf)
        v = cbuf[...]
        return jnp.sum(jnp.where(iota16 == col, v, 0))

    locs = (loc0, loc1)

    def run_direction(src_table, dArr, sArr, dir_idx, out_hbm):
        pltpu.sync_copy(zrows, acc.at[pl.ds(s * ZBLK, ZBLK)])
        plsc.subcore_barrier()

        col = 2 * dir_idx + c
        nA = get_count(s, col)
        nB = get_count(s + NS, col)
        T = nA + nB

        def chunk_src(t, arr):
            inA = t < nA
            slot = jnp.where(inA, s, s + NS)
            chv = jnp.where(inA, t, t - nA)
            return arr.at[c, slot, pl.ds(chv * CHUNK, CHUNK)]

        def issue_idx(t, b):
            pltpu.async_copy(chunk_src(t, dArr), dstb.at[b], sem_i[b])
            pltpu.async_copy(chunk_src(t, sArr), srcb.at[b], sem_i[b])

        def wait_idx(t, b):
            pltpu.make_async_copy(chunk_src(t, dArr), dstb.at[b],
                                  sem_i[b]).wait()
            pltpu.make_async_copy(chunk_src(t, sArr), srcb.at[b],
                                  sem_i[b]).wait()

        def issue_gather(b):
            pltpu.async_copy(src_table.at[srcb.at[b]], gbuf.at[b], sem_g[b])

        def wait_gather(b):
            pltpu.make_async_copy(src_table.at[srcb.at[b]], gbuf.at[b],
                                  sem_g[b]).wait()

        def do_scatters(b):
            lref = locs[b]
            for j in range(CHUNK // LANES):
                d = dstb[b, pl.ds(j * LANES, LANES)]
                keep = jnp.logical_and(d >= lo, d < lo + HALF)
                loc = jnp.where(keep, d - lo, TRASH + (d & 63))
                plsc.store_scatter(lref, [j * LANES + iota16], loc)
            pltpu.async_copy(gbuf.at[b], acc.at[lref], sem_s[b], add=True)

        def wait_scatters(b):
            pltpu.make_async_copy(gbuf.at[b], acc.at[locs[b]],
                                  sem_s[b]).wait()

        @pl.when(T > 0)
        def _():
            issue_idx(0, 0)
            wait_idx(0, 0)
            issue_gather(0)

        @pl.when(T > 1)
        def _():
            issue_idx(1, 1)

        @pl.loop(0, (T + 1) // 2)
        def _(p):
            for b in (0, 1):
                t = p * 2 + b

                @pl.when(jnp.logical_and(t >= 1, t <= T - 1))
                def _():
                    wait_scatters(1 - b)

                @pl.when(t + 1 < T)
                def _():
                    wait_idx(t + 1, 1 - b)
                    issue_gather(1 - b)

                @pl.when(t < T)
                def _():
                    wait_gather(b)
                    do_scatters(b)

                @pl.when(t + 2 < T)
                def _():
                    issue_idx(t + 2, b)

        @pl.when(T > 0)
        def _():
            @pl.when((T - 1) % 2 == 0)
            def _():
                wait_scatters(0)

            @pl.when((T - 1) % 2 == 1)
            def _():
                wait_scatters(1)

        plsc.subcore_barrier()

        @pl.loop(s, HALF // COPY_BLK, step=NS)
        def _(ch):
            pltpu.sync_copy(acc.at[pl.ds(ch * COPY_BLK, COPY_BLK)],
                            out_hbm.at[pl.ds(lo + ch * COPY_BLK, COPY_BLK)])
        plsc.subcore_barrier()

    run_direction(si, dstU, srcU, 0, accu)
    run_direction(su, dstI, srcI, 1, acci)


def _spmm_pair(zrows, si, su, dstU, srcU, dstI, srcI, cnt):
    f = pl.kernel(
        _spmm_body,
        out_type=[jax.ShapeDtypeStruct((N_USERS, D), jnp.float32),
                  jax.ShapeDtypeStruct((N_ITEMS, D), jnp.float32)],
        mesh=_sc_mesh(),
        scratch_types=[
            pltpu.VMEM((2, CHUNK), jnp.int32),
            pltpu.VMEM((2, CHUNK), jnp.int32),
            pltpu.VMEM((2, CHUNK, D), jnp.float32),
            pltpu.VMEM((LANES,), jnp.int32),
            pltpu.VMEM((CHUNK,), jnp.int32),
            pltpu.VMEM((CHUNK,), jnp.int32),
            pltpu.VMEM_SHARED((ACC_ROWS, D), jnp.float32),
            pltpu.SemaphoreType.DMA,
            pltpu.SemaphoreType.DMA,
            pltpu.SemaphoreType.DMA,
            pltpu.SemaphoreType.DMA,
            pltpu.SemaphoreType.DMA,
            pltpu.SemaphoreType.DMA,
        ],
        compiler_params=_SC_PARAMS,
    )
    return f(zrows, si, su, dstU, srcU, dstI, srcI, cnt)


def _final_gather_body(tabs, idxs, outs, idxb, gb, sem):
    c = lax.axis_index("c")
    s = lax.axis_index("s")
    w = s * NC + c
    n = BATCH // NW
    base = w * n
    for r, idx_arr in enumerate(idxs):
        pltpu.sync_copy(idx_arr.at[pl.ds(base, n)], idxb.at[r])
    for t in range(4 * len(idxs)):
        r = t // 4
        tab = tabs[(0 if r == 0 else 1) * 4 + t % 4]
        pltpu.sync_copy(tab.at[idxb.at[r]], gb)
        pltpu.sync_copy(gb, outs[t].at[pl.ds(base, n)])


def _final_gather(gu, gi, users, pos_item, neg_item):
    def body(u0, u1, u2, u3, i0, i1, i2, i3, users_, pos_, neg_, *rest):
        outs = rest[:12]
        idxb, gb, sem = rest[12:]
        _final_gather_body((u0, u1, u2, u3, i0, i1, i2, i3),
                           (users_, pos_, neg_), outs, idxb, gb, sem)

    f = pl.kernel(
        body,
        out_type=[jax.ShapeDtypeStruct((BATCH, D), jnp.float32)] * 12,
        mesh=_sc_mesh(),
        scratch_types=[
            pltpu.VMEM((3, BATCH // NW), jnp.int32),
            pltpu.VMEM((BATCH // NW, D), jnp.float32),
            pltpu.SemaphoreType.DMA,
        ],
        compiler_params=_SC_PARAMS,
    )
    return f(*gu, *gi, users, pos_item, neg_item)


def _combine_body(accu_ref, oldu_ref, du_ref, acci_ref, oldi_ref, di_ref,
                  newu_ref, su_ref, newi_ref, si_ref):
    au = jnp.sqrt(du_ref[...])
    new_u = au * accu_ref[...] + du_ref[...] * oldu_ref[...]
    newu_ref[...] = new_u
    su_ref[...] = au * new_u
    ai = jnp.sqrt(di_ref[...])
    new_i = ai * acci_ref[...] + di_ref[...] * oldi_ref[...]
    newi_ref[...] = new_i
    si_ref[...] = ai * new_i


def _combine(acc_u, old_u, d_u, acc_i, old_i, d_i):
    n = acc_u.shape[0]
    spec_t = pl.BlockSpec((ROW_BLK, D), lambda i: (i, 0))
    spec_v = pl.BlockSpec((ROW_BLK, 1), lambda i: (i, 0))
    return pl.pallas_call(
        _combine_body,
        grid=(n // ROW_BLK,),
        in_specs=[spec_t, spec_t, spec_v, spec_t, spec_t, spec_v],
        out_specs=[spec_t, spec_t, spec_t, spec_t],
        out_shape=[jax.ShapeDtypeStruct((n, D), jnp.float32)] * 4,
    )(acc_u, old_u, d_u, acc_i, old_i, d_i)


def _prescale_body(oldu_ref, du_ref, oldi_ref, di_ref, su_ref, si_ref):
    su_ref[...] = jnp.sqrt(du_ref[...]) * oldu_ref[...]
    si_ref[...] = jnp.sqrt(di_ref[...]) * oldi_ref[...]


def _prescale(old_u, d_u, old_i, d_i):
    n = old_u.shape[0]
    spec_t = pl.BlockSpec((ROW_BLK, D), lambda i: (i, 0))
    spec_v = pl.BlockSpec((ROW_BLK, 1), lambda i: (i, 0))
    return pl.pallas_call(
        _prescale_body,
        grid=(n // ROW_BLK,),
        in_specs=[spec_t, spec_v, spec_t, spec_v],
        out_specs=[spec_t, spec_t],
        out_shape=[jax.ShapeDtypeStruct((n, D), jnp.float32)] * 2,
    )(old_u, d_u, old_i, d_i)


def kernel(emb_user, emb_item, ui_vals, iu_vals, d_users, d_items,
           users, pos_item, neg_item, rows, cols):
    gu = [emb_user]
    gi = [emb_item]
    zrows = jnp.zeros((ZBLK, D), jnp.float32)
    dstU, srcU, dstI, srcI, cnt = _bucket(rows, cols)
    su, si = _prescale(emb_user, d_users, emb_item, d_items)
    for _ in range(LAYERS):
        acc_u, acc_i = _spmm_pair(zrows, si, su, dstU, srcU, dstI, srcI, cnt)
        new_u, su, new_i, si = _combine(acc_u, gu[-1], d_users,
                                        acc_i, gi[-1], d_items)
        gu.append(new_u)
        gi.append(new_i)

    pieces = _final_gather(gu, gi, users, pos_item, neg_item)
    u_emb = jnp.concatenate(pieces[0:4], axis=1)
    pos_emb = jnp.concatenate(pieces[4:8], axis=1)
    neg_emb = jnp.concatenate(pieces[8:12], axis=1)
    return u_emb, pos_emb, neg_emb


# speedup vs baseline: 7.6823x; 1.0765x over previous
import jax
import jax.numpy as jnp
from jax import lax
from jax.experimental import pallas as pl
from jax.experimental.pallas import tpu as pltpu
from jax.experimental.pallas import tpu_sc as plsc

N_USERS = 50000
N_ITEMS = 50000
NNZ = 800000
D = 64
LAYERS = 3
BATCH = 4096

_SC_PARAMS = pltpu.CompilerParams(needs_layout_passes=False,
                                  use_tc_tiling_on_sc=False)

NC = 2
NS = 16
NW = NC * NS
LANES = 16

HALF = N_USERS // NC
TRASH = HALF
ACC_ROWS = 25200
COPY_BLK = 200
ZBLK = ACC_ROWS // NS
CHUNK = 128
NCHT = NNZ // CHUNK
NCHW = 196
SLOT_CAP = NCHW * CHUNK
SENTINEL = 2 * N_USERS

ROW_BLK = 2000


def _sc_mesh():
    return plsc.VectorSubcoreMesh(core_axis_name="c", subcore_axis_name="s",
                                  num_cores=NC, num_subcores=NS)


def _bucket_body(rows, cols, dstU, srcU, dstI, srcI, cnt,
                 ebufd, ebufs, sd0, ss0, sd1, ss1, cntb, sem0, sem1):
    c = lax.axis_index("c")
    s = lax.axis_index("s")
    w = s * NC + c
    nch = 195 + jnp.where(w < 10, 1, 0)
    iota16 = lax.iota(jnp.int32, LANES)
    sems = (sem0, sem1)

    def pass_dir(dst_arr, src_arr, outD, outS, dir_idx, cv_in):
        def issue(j, b):
            off = (j * NW + w) * CHUNK
            pltpu.async_copy(dst_arr.at[pl.ds(off, CHUNK)], ebufd.at[b],
                             sems[b])
            pltpu.async_copy(src_arr.at[pl.ds(off, CHUNK)], ebufs.at[b],
                             sems[b])

        def wait(j, b):
            off = (j * NW + w) * CHUNK
            pltpu.make_async_copy(dst_arr.at[pl.ds(off, CHUNK)],
                                  ebufd.at[b], sems[b]).wait()
            pltpu.make_async_copy(src_arr.at[pl.ds(off, CHUNK)],
                                  ebufs.at[b], sems[b]).wait()

        issue(0, 0)

        def pair(p, carry):
            off0, off1 = carry
            for b in (0, 1):
                j = p * 2 + b
                valid = j < nch

                @pl.when(valid)
                def _():
                    wait(j, b)

                @pl.when(j + 1 < nch)
                def _():
                    issue(j + 1, 1 - b)

                for q in range(CHUNK // LANES):
                    d = ebufd[b, pl.ds(q * LANES, LANES)]
                    sv = ebufs[b, pl.ds(q * LANES, LANES)]
                    m0 = jnp.logical_and(d < HALF, valid)
                    m1 = jnp.logical_and(d >= HALF, valid)
                    i0 = m0.astype(jnp.int32)
                    cs0 = plsc.cumsum(i0)
                    p0 = off0 + cs0 - 1
                    p1 = off1 + iota16 - cs0
                    plsc.store_scatter(sd0, [p0], d, mask=m0)
                    plsc.store_scatter(ss0, [p0], sv, mask=m0)
                    plsc.store_scatter(sd1, [p1], d, mask=m1)
                    plsc.store_scatter(ss1, [p1], sv, mask=m1)
                    s0 = jnp.sum(i0)
                    off0 = off0 + s0
                    off1 = off1 + jnp.where(valid, LANES - s0, 0)
            return off0, off1

        n0, n1 = lax.fori_loop(0, NCHW // 2, pair,
                               (jnp.int32(0), jnp.int32(0)))

        def finish(n, sdst, ssrc, outDref, outSref):
            pend = jnp.minimum((n + CHUNK - 1) & (-CHUNK), SLOT_CAP)
            sent = jnp.full((LANES,), SENTINEL, jnp.int32)
            zero = jnp.zeros((LANES,), jnp.int32)
            for t in range(CHUNK // LANES):
                pos = n + t * LANES + iota16
                m = pos < pend
                plsc.store_scatter(sdst, [pos], sent, mask=m)
                plsc.store_scatter(ssrc, [pos], zero, mask=m)
            pltpu.sync_copy(sdst, outDref)
            pltpu.sync_copy(ssrc, outSref)
            return pend // CHUNK

        nch0 = finish(n0, sd0, ss0, outD.at[0, w], outS.at[0, w])
        nch1 = finish(n1, sd1, ss1, outD.at[1, w], outS.at[1, w])
        cv = jnp.where(iota16 == 2 * dir_idx, nch0,
                       jnp.where(iota16 == 2 * dir_idx + 1, nch1, cv_in))
        return cv

    cv = jnp.zeros((LANES,), jnp.int32)
    cv = pass_dir(rows, cols, dstU, srcU, 0, cv)
    cv = pass_dir(cols, rows, dstI, srcI, 1, cv)
    cntb[...] = cv
    pltpu.sync_copy(cntb, cnt.at[w])


def _bucket(rows, cols):
    slot = jax.ShapeDtypeStruct((2, NW, SLOT_CAP), jnp.int32)
    f = pl.kernel(
        _bucket_body,
        out_type=[slot, slot, slot, slot,
                  jax.ShapeDtypeStruct((NW, LANES), jnp.int32)],
        mesh=_sc_mesh(),
        scratch_types=[
            pltpu.VMEM((2, CHUNK), jnp.int32),
            pltpu.VMEM((2, CHUNK), jnp.int32),
            pltpu.VMEM((SLOT_CAP,), jnp.int32),
            pltpu.VMEM((SLOT_CAP,), jnp.int32),
            pltpu.VMEM((SLOT_CAP,), jnp.int32),
            pltpu.VMEM((SLOT_CAP,), jnp.int32),
            pltpu.VMEM((LANES,), jnp.int32),
            pltpu.SemaphoreType.DMA,
            pltpu.SemaphoreType.DMA,
        ],
        compiler_params=_SC_PARAMS,
    )
    return f(rows, cols)


def _spmm_body(zrows, si, su, dstU, srcU, dstI, srcI, cnt, accu, acci,
               dstb, srcb, gbuf, cbuf, loc0, loc1, loc2, acc,
               sem_i0, sem_i1, sem_i2, sem_g0, sem_g1, sem_g2,
               sem_s0, sem_s1, sem_s2):
    c = lax.axis_index("c")
    s = lax.axis_index("s")
    lo = c * HALF
    sem_i = (sem_i0, sem_i1, sem_i2)
    sem_g = (sem_g0, sem_g1, sem_g2)
    sem_s = (sem_s0, sem_s1, sem_s2)

    iota16 = lax.iota(jnp.int32, LANES)

    def get_count(slot, col):
        pltpu.sync_copy(cnt.at[slot], cbuf)
        v = cbuf[...]
        return jnp.sum(jnp.where(iota16 == col, v, 0))

    locs = (loc0, loc1, loc2)

    def run_direction(src_table, dArr, sArr, dir_idx, out_hbm):
        pltpu.sync_copy(zrows, acc.at[pl.ds(s * ZBLK, ZBLK)])
        plsc.subcore_barrier()

        col = 2 * dir_idx + c
        nA = get_count(s, col)
        nB = get_count(s + NS, col)
        T = nA + nB

        def chunk_src(t, arr):
            inA = t < nA
            slot = jnp.where(inA, s, s + NS)
            chv = jnp.where(inA, t, t - nA)
            return arr.at[c, slot, pl.ds(chv * CHUNK, CHUNK)]

        def issue_idx(t, b):
            pltpu.async_copy(chunk_src(t, dArr), dstb.at[b], sem_i[b])
            pltpu.async_copy(chunk_src(t, sArr), srcb.at[b], sem_i[b])

        def wait_idx(t, b):
            pltpu.make_async_copy(chunk_src(t, dArr), dstb.at[b],
                                  sem_i[b]).wait()
            pltpu.make_async_copy(chunk_src(t, sArr), srcb.at[b],
                                  sem_i[b]).wait()

        def issue_gather(b):
            pltpu.async_copy(src_table.at[srcb.at[b]], gbuf.at[b], sem_g[b])

        def wait_gather(b):
            pltpu.make_async_copy(src_table.at[srcb.at[b]], gbuf.at[b],
                                  sem_g[b]).wait()

        def do_scatters(b):
            lref = locs[b]
            for j in range(CHUNK // LANES):
                d = dstb[b, pl.ds(j * LANES, LANES)]
                keep = jnp.logical_and(d >= lo, d < lo + HALF)
                loc = jnp.where(keep, d - lo, TRASH + (d & 63))
                plsc.store_scatter(lref, [j * LANES + iota16], loc)
            pltpu.async_copy(gbuf.at[b], acc.at[lref], sem_s[b], add=True)

        def wait_scatters(b):
            pltpu.make_async_copy(gbuf.at[b], acc.at[locs[b]],
                                  sem_s[b]).wait()

        for t0 in (0, 1, 2):
            @pl.when(t0 < T)
            def _():
                issue_idx(t0, t0)

        for t0 in (0, 1):
            @pl.when(t0 < T)
            def _():
                wait_idx(t0, t0)
                issue_gather(t0)

        @pl.loop(0, (T + 3) // 3)
        def _(p):
            for b in (0, 1, 2):
                t = p * 3 + b

                @pl.when(jnp.logical_and(t >= 1, t - 1 < T))
                def _():
                    wait_scatters((b + 2) % 3)

                @pl.when(t + 2 < T)
                def _():
                    wait_idx(t + 2, (b + 2) % 3)
                    issue_gather((b + 2) % 3)

                @pl.when(t < T)
                def _():
                    wait_gather(b)
                    do_scatters(b)

                @pl.when(t + 3 < T)
                def _():
                    issue_idx(t + 3, b)

        plsc.subcore_barrier()

        @pl.loop(s, HALF // COPY_BLK, step=NS)
        def _(ch):
            pltpu.sync_copy(acc.at[pl.ds(ch * COPY_BLK, COPY_BLK)],
                            out_hbm.at[pl.ds(lo + ch * COPY_BLK, COPY_BLK)])
        plsc.subcore_barrier()

    run_direction(si, dstU, srcU, 0, accu)
    run_direction(su, dstI, srcI, 1, acci)


def _spmm_pair(zrows, si, su, dstU, srcU, dstI, srcI, cnt):
    f = pl.kernel(
        _spmm_body,
        out_type=[jax.ShapeDtypeStruct((N_USERS, D), jnp.float32),
                  jax.ShapeDtypeStruct((N_ITEMS, D), jnp.float32)],
        mesh=_sc_mesh(),
        scratch_types=[
            pltpu.VMEM((3, CHUNK), jnp.int32),
            pltpu.VMEM((3, CHUNK), jnp.int32),
            pltpu.VMEM((3, CHUNK, D), jnp.float32),
            pltpu.VMEM((LANES,), jnp.int32),
            pltpu.VMEM((CHUNK,), jnp.int32),
            pltpu.VMEM((CHUNK,), jnp.int32),
            pltpu.VMEM((CHUNK,), jnp.int32),
            pltpu.VMEM_SHARED((ACC_ROWS, D), jnp.float32),
        ] + [pltpu.SemaphoreType.DMA] * 9,
        compiler_params=_SC_PARAMS,
    )
    return f(zrows, si, su, dstU, srcU, dstI, srcI, cnt)


def _final_gather_body(tabs, idxs, outs, idxb, gb, sem):
    c = lax.axis_index("c")
    s = lax.axis_index("s")
    w = s * NC + c
    n = BATCH // NW
    base = w * n
    for r, idx_arr in enumerate(idxs):
        pltpu.sync_copy(idx_arr.at[pl.ds(base, n)], idxb.at[r])
    for t in range(4 * len(idxs)):
        r = t // 4
        tab = tabs[(0 if r == 0 else 1) * 4 + t % 4]
        pltpu.sync_copy(tab.at[idxb.at[r]], gb)
        pltpu.sync_copy(gb, outs[t].at[pl.ds(base, n)])


def _final_gather(gu, gi, users, pos_item, neg_item):
    def body(u0, u1, u2, u3, i0, i1, i2, i3, users_, pos_, neg_, *rest):
        outs = rest[:12]
        idxb, gb, sem = rest[12:]
        _final_gather_body((u0, u1, u2, u3, i0, i1, i2, i3),
                           (users_, pos_, neg_), outs, idxb, gb, sem)

    f = pl.kernel(
        body,
        out_type=[jax.ShapeDtypeStruct((BATCH, D), jnp.float32)] * 12,
        mesh=_sc_mesh(),
        scratch_types=[
            pltpu.VMEM((3, BATCH // NW), jnp.int32),
            pltpu.VMEM((BATCH // NW, D), jnp.float32),
            pltpu.SemaphoreType.DMA,
        ],
        compiler_params=_SC_PARAMS,
    )
    return f(*gu, *gi, users, pos_item, neg_item)


def _combine_body(accu_ref, oldu_ref, du_ref, acci_ref, oldi_ref, di_ref,
                  newu_ref, su_ref, newi_ref, si_ref):
    au = jnp.sqrt(du_ref[...])
    new_u = au * accu_ref[...] + du_ref[...] * oldu_ref[...]
    newu_ref[...] = new_u
    su_ref[...] = au * new_u
    ai = jnp.sqrt(di_ref[...])
    new_i = ai * acci_ref[...] + di_ref[...] * oldi_ref[...]
    newi_ref[...] = new_i
    si_ref[...] = ai * new_i


def _combine(acc_u, old_u, d_u, acc_i, old_i, d_i):
    n = acc_u.shape[0]
    spec_t = pl.BlockSpec((ROW_BLK, D), lambda i: (i, 0))
    spec_v = pl.BlockSpec((ROW_BLK, 1), lambda i: (i, 0))
    return pl.pallas_call(
        _combine_body,
        grid=(n // ROW_BLK,),
        in_specs=[spec_t, spec_t, spec_v, spec_t, spec_t, spec_v],
        out_specs=[spec_t, spec_t, spec_t, spec_t],
        out_shape=[jax.ShapeDtypeStruct((n, D), jnp.float32)] * 4,
    )(acc_u, old_u, d_u, acc_i, old_i, d_i)


def _prescale_body(oldu_ref, du_ref, oldi_ref, di_ref, su_ref, si_ref):
    su_ref[...] = jnp.sqrt(du_ref[...]) * oldu_ref[...]
    si_ref[...] = jnp.sqrt(di_ref[...]) * oldi_ref[...]


def _prescale(old_u, d_u, old_i, d_i):
    n = old_u.shape[0]
    spec_t = pl.BlockSpec((ROW_BLK, D), lambda i: (i, 0))
    spec_v = pl.BlockSpec((ROW_BLK, 1), lambda i: (i, 0))
    return pl.pallas_call(
        _prescale_body,
        grid=(n // ROW_BLK,),
        in_specs=[spec_t, spec_v, spec_t, spec_v],
        out_specs=[spec_t, spec_t],
        out_shape=[jax.ShapeDtypeStruct((n, D), jnp.float32)] * 2,
    )(old_u, d_u, old_i, d_i)


def kernel(emb_user, emb_item, ui_vals, iu_vals, d_users, d_items,
           users, pos_item, neg_item, rows, cols):
    gu = [emb_user]
    gi = [emb_item]
    zrows = jnp.zeros((ZBLK, D), jnp.float32)
    dstU, srcU, dstI, srcI, cnt = _bucket(rows, cols)
    su, si = _prescale(emb_user, d_users, emb_item, d_items)
    for _ in range(LAYERS):
        acc_u, acc_i = _spmm_pair(zrows, si, su, dstU, srcU, dstI, srcI, cnt)
        new_u, su, new_i, si = _combine(acc_u, gu[-1], d_users,
                                        acc_i, gi[-1], d_items)
        gu.append(new_u)
        gi.append(new_i)

    pieces = _final_gather(gu, gi, users, pos_item, neg_item)
    u_emb = jnp.concatenate(pieces[0:4], axis=1)
    pos_emb = jnp.concatenate(pieces[4:8], axis=1)
    neg_emb = jnp.concatenate(pieces[8:12], axis=1)
    return u_emb, pos_emb, neg_emb


# speedup vs baseline: 8.0679x; 1.0502x over previous
import jax
import jax.numpy as jnp
from jax import lax
from jax.experimental import pallas as pl
from jax.experimental.pallas import tpu as pltpu
from jax.experimental.pallas import tpu_sc as plsc

N_USERS = 50000
N_ITEMS = 50000
NNZ = 800000
D = 64
LAYERS = 3
BATCH = 4096

_SC_PARAMS = pltpu.CompilerParams(needs_layout_passes=False,
                                  use_tc_tiling_on_sc=False)

NC = 2
NS = 16
NW = NC * NS
LANES = 16

HALF = N_USERS // NC
TRASH = HALF
ACC_ROWS = 25200
COPY_BLK = 200
ZBLK = ACC_ROWS // NS
CHUNK = 128
NCHT = NNZ // CHUNK
NCHW = 196
SLOT_CAP = NCHW * CHUNK
SENTINEL = 2 * N_USERS

ROW_BLK = 2000


def _sc_mesh():
    return plsc.VectorSubcoreMesh(core_axis_name="c", subcore_axis_name="s",
                                  num_cores=NC, num_subcores=NS)


def _bucket_body(rows, cols, dstU, srcU, dstI, srcI, cnt,
                 ebufd, ebufs, sd0, ss0, sd1, ss1, cntb,
                 sem0, sem1, sem2, sem3):
    c = lax.axis_index("c")
    s = lax.axis_index("s")
    w = s * NC + c
    nch = 195 + jnp.where(w < 10, 1, 0)
    iota16 = lax.iota(jnp.int32, LANES)
    sems = (sem0, sem1, sem2, sem3)

    def pass_dir(dst_arr, src_arr, outD, outS, dir_idx, cv_in):
        def issue(j, b):
            off = (j * NW + w) * CHUNK
            pltpu.async_copy(dst_arr.at[pl.ds(off, CHUNK)], ebufd.at[b],
                             sems[b])
            pltpu.async_copy(src_arr.at[pl.ds(off, CHUNK)], ebufs.at[b],
                             sems[b])

        def wait(j, b):
            off = (j * NW + w) * CHUNK
            pltpu.make_async_copy(dst_arr.at[pl.ds(off, CHUNK)],
                                  ebufd.at[b], sems[b]).wait()
            pltpu.make_async_copy(src_arr.at[pl.ds(off, CHUNK)],
                                  ebufs.at[b], sems[b]).wait()

        for j0 in (0, 1, 2):
            @pl.when(j0 < nch)
            def _():
                issue(j0, j0)

        def pair(p, carry):
            off0, off1 = carry
            for b in (0, 1, 2, 3):
                j = p * 4 + b
                valid = j < nch

                @pl.when(valid)
                def _():
                    wait(j, b)

                @pl.when(j + 3 < nch)
                def _():
                    issue(j + 3, (b + 3) % 4)

                for q in range(CHUNK // LANES):
                    d = ebufd[b, pl.ds(q * LANES, LANES)]
                    sv = ebufs[b, pl.ds(q * LANES, LANES)]
                    m0 = jnp.logical_and(d < HALF, valid)
                    m1 = jnp.logical_and(d >= HALF, valid)
                    i0 = m0.astype(jnp.int32)
                    cs0 = plsc.cumsum(i0)
                    p0 = off0 + cs0 - 1
                    p1 = off1 + iota16 - cs0
                    plsc.store_scatter(sd0, [p0], d, mask=m0)
                    plsc.store_scatter(ss0, [p0], sv, mask=m0)
                    plsc.store_scatter(sd1, [p1], d, mask=m1)
                    plsc.store_scatter(ss1, [p1], sv, mask=m1)
                    s0 = jnp.sum(i0)
                    off0 = off0 + s0
                    off1 = off1 + jnp.where(valid, LANES - s0, 0)
            return off0, off1

        n0, n1 = lax.fori_loop(0, NCHW // 4, pair,
                               (jnp.int32(0), jnp.int32(0)))

        def finish(n, sdst, ssrc, outDref, outSref):
            pend = jnp.minimum((n + CHUNK - 1) & (-CHUNK), SLOT_CAP)
            sent = jnp.full((LANES,), SENTINEL, jnp.int32)
            zero = jnp.zeros((LANES,), jnp.int32)
            for t in range(CHUNK // LANES):
                pos = n + t * LANES + iota16
                m = pos < pend
                plsc.store_scatter(sdst, [pos], sent, mask=m)
                plsc.store_scatter(ssrc, [pos], zero, mask=m)
            pltpu.sync_copy(sdst, outDref)
            pltpu.sync_copy(ssrc, outSref)
            return pend // CHUNK

        nch0 = finish(n0, sd0, ss0, outD.at[0, w], outS.at[0, w])
        nch1 = finish(n1, sd1, ss1, outD.at[1, w], outS.at[1, w])
        cv = jnp.where(iota16 == 2 * dir_idx, nch0,
                       jnp.where(iota16 == 2 * dir_idx + 1, nch1, cv_in))
        return cv

    cv = jnp.zeros((LANES,), jnp.int32)
    cv = pass_dir(rows, cols, dstU, srcU, 0, cv)
    cv = pass_dir(cols, rows, dstI, srcI, 1, cv)
    cntb[...] = cv
    pltpu.sync_copy(cntb, cnt.at[w])


def _bucket(rows, cols):
    slot = jax.ShapeDtypeStruct((2, NW, SLOT_CAP), jnp.int32)
    f = pl.kernel(
        _bucket_body,
        out_type=[slot, slot, slot, slot,
                  jax.ShapeDtypeStruct((NW, LANES), jnp.int32)],
        mesh=_sc_mesh(),
        scratch_types=[
            pltpu.VMEM((4, CHUNK), jnp.int32),
            pltpu.VMEM((4, CHUNK), jnp.int32),
            pltpu.VMEM((SLOT_CAP,), jnp.int32),
            pltpu.VMEM((SLOT_CAP,), jnp.int32),
            pltpu.VMEM((SLOT_CAP,), jnp.int32),
            pltpu.VMEM((SLOT_CAP,), jnp.int32),
            pltpu.VMEM((LANES,), jnp.int32),
        ] + [pltpu.SemaphoreType.DMA] * 4,
        compiler_params=_SC_PARAMS,
    )
    return f(rows, cols)


def _spmm_body(zrows, si, su, dstU, srcU, dstI, srcI, cnt, accu, acci,
               dstb, srcb, gbuf, cbuf, loc0, loc1, loc2, acc,
               sem_i0, sem_i1, sem_i2, sem_g0, sem_g1, sem_g2,
               sem_s0, sem_s1, sem_s2):
    c = lax.axis_index("c")
    s = lax.axis_index("s")
    lo = c * HALF
    sem_i = (sem_i0, sem_i1, sem_i2)
    sem_g = (sem_g0, sem_g1, sem_g2)
    sem_s = (sem_s0, sem_s1, sem_s2)

    iota16 = lax.iota(jnp.int32, LANES)

    def get_count(slot, col):
        pltpu.sync_copy(cnt.at[slot], cbuf)
        v = cbuf[...]
        return jnp.sum(jnp.where(iota16 == col, v, 0))

    locs = (loc0, loc1, loc2)

    def run_direction(src_table, dArr, sArr, dir_idx, out_hbm):
        pltpu.sync_copy(zrows, acc.at[pl.ds(s * ZBLK, ZBLK)])
        plsc.subcore_barrier()

        col = 2 * dir_idx + c
        nA = get_count(s, col)
        nB = get_count(s + NS, col)
        T = nA + nB

        def chunk_src(t, arr):
            inA = t < nA
            slot = jnp.where(inA, s, s + NS)
            chv = jnp.where(inA, t, t - nA)
            return arr.at[c, slot, pl.ds(chv * CHUNK, CHUNK)]

        def issue_idx(t, b):
            pltpu.async_copy(chunk_src(t, dArr), dstb.at[b], sem_i[b])
            pltpu.async_copy(chunk_src(t, sArr), srcb.at[b], sem_i[b])

        def wait_idx(t, b):
            pltpu.make_async_copy(chunk_src(t, dArr), dstb.at[b],
                                  sem_i[b]).wait()
            pltpu.make_async_copy(chunk_src(t, sArr), srcb.at[b],
                                  sem_i[b]).wait()

        def issue_gather(b):
            pltpu.async_copy(src_table.at[srcb.at[b]], gbuf.at[b], sem_g[b])

        def wait_gather(b):
            pltpu.make_async_copy(src_table.at[srcb.at[b]], gbuf.at[b],
                                  sem_g[b]).wait()

        def do_scatters(b):
            lref = locs[b]
            for j in range(CHUNK // LANES):
                d = dstb[b, pl.ds(j * LANES, LANES)]
                keep = jnp.logical_and(d >= lo, d < lo + HALF)
                loc = jnp.where(keep, d - lo, TRASH + (d & 63))
                plsc.store_scatter(lref, [j * LANES + iota16], loc)
            pltpu.async_copy(gbuf.at[b], acc.at[lref], sem_s[b], add=True)

        def wait_scatters(b):
            pltpu.make_async_copy(gbuf.at[b], acc.at[locs[b]],
                                  sem_s[b]).wait()

        for t0 in (0, 1, 2):
            @pl.when(t0 < T)
            def _():
                issue_idx(t0, t0)

        for t0 in (0, 1):
            @pl.when(t0 < T)
            def _():
                wait_idx(t0, t0)
                issue_gather(t0)

        @pl.loop(0, (T + 3) // 3)
        def _(p):
            for b in (0, 1, 2):
                t = p * 3 + b

                @pl.when(jnp.logical_and(t >= 1, t - 1 < T))
                def _():
                    wait_scatters((b + 2) % 3)

                @pl.when(t + 2 < T)
                def _():
                    wait_idx(t + 2, (b + 2) % 3)
                    issue_gather((b + 2) % 3)

                @pl.when(t < T)
                def _():
                    wait_gather(b)
                    do_scatters(b)

                @pl.when(t + 3 < T)
                def _():
                    issue_idx(t + 3, b)

        plsc.subcore_barrier()

        @pl.loop(s, HALF // COPY_BLK, step=NS)
        def _(ch):
            pltpu.sync_copy(acc.at[pl.ds(ch * COPY_BLK, COPY_BLK)],
                            out_hbm.at[pl.ds(lo + ch * COPY_BLK, COPY_BLK)])
        plsc.subcore_barrier()

    run_direction(si, dstU, srcU, 0, accu)
    run_direction(su, dstI, srcI, 1, acci)


def _spmm_pair(zrows, si, su, dstU, srcU, dstI, srcI, cnt):
    f = pl.kernel(
        _spmm_body,
        out_type=[jax.ShapeDtypeStruct((N_USERS, D), jnp.float32),
                  jax.ShapeDtypeStruct((N_ITEMS, D), jnp.float32)],
        mesh=_sc_mesh(),
        scratch_types=[
            pltpu.VMEM((3, CHUNK), jnp.int32),
            pltpu.VMEM((3, CHUNK), jnp.int32),
            pltpu.VMEM((3, CHUNK, D), jnp.float32),
            pltpu.VMEM((LANES,), jnp.int32),
            pltpu.VMEM((CHUNK,), jnp.int32),
            pltpu.VMEM((CHUNK,), jnp.int32),
            pltpu.VMEM((CHUNK,), jnp.int32),
            pltpu.VMEM_SHARED((ACC_ROWS, D), jnp.float32),
        ] + [pltpu.SemaphoreType.DMA] * 9,
        compiler_params=_SC_PARAMS,
    )
    return f(zrows, si, su, dstU, srcU, dstI, srcI, cnt)


def _final_gather_body(tabs, idxs, outs, idxb, gb, sem):
    c = lax.axis_index("c")
    s = lax.axis_index("s")
    w = s * NC + c
    n = BATCH // NW
    base = w * n
    for r, idx_arr in enumerate(idxs):
        pltpu.sync_copy(idx_arr.at[pl.ds(base, n)], idxb.at[r])
    for t in range(4 * len(idxs)):
        r = t // 4
        tab = tabs[(0 if r == 0 else 1) * 4 + t % 4]
        pltpu.sync_copy(tab.at[idxb.at[r]], gb)
        pltpu.sync_copy(gb, outs[t].at[pl.ds(base, n)])


def _final_gather(gu, gi, users, pos_item, neg_item):
    def body(u0, u1, u2, u3, i0, i1, i2, i3, users_, pos_, neg_, *rest):
        outs = rest[:12]
        idxb, gb, sem = rest[12:]
        _final_gather_body((u0, u1, u2, u3, i0, i1, i2, i3),
                           (users_, pos_, neg_), outs, idxb, gb, sem)

    f = pl.kernel(
        body,
        out_type=[jax.ShapeDtypeStruct((BATCH, D), jnp.float32)] * 12,
        mesh=_sc_mesh(),
        scratch_types=[
            pltpu.VMEM((3, BATCH // NW), jnp.int32),
            pltpu.VMEM((BATCH // NW, D), jnp.float32),
            pltpu.SemaphoreType.DMA,
        ],
        compiler_params=_SC_PARAMS,
    )
    return f(*gu, *gi, users, pos_item, neg_item)


def _combine_body(accu_ref, oldu_ref, du_ref, acci_ref, oldi_ref, di_ref,
                  newu_ref, su_ref, newi_ref, si_ref):
    au = jnp.sqrt(du_ref[...])
    new_u = au * accu_ref[...] + du_ref[...] * oldu_ref[...]
    newu_ref[...] = new_u
    su_ref[...] = au * new_u
    ai = jnp.sqrt(di_ref[...])
    new_i = ai * acci_ref[...] + di_ref[...] * oldi_ref[...]
    newi_ref[...] = new_i
    si_ref[...] = ai * new_i


def _combine(acc_u, old_u, d_u, acc_i, old_i, d_i):
    n = acc_u.shape[0]
    spec_t = pl.BlockSpec((ROW_BLK, D), lambda i: (i, 0))
    spec_v = pl.BlockSpec((ROW_BLK, 1), lambda i: (i, 0))
    return pl.pallas_call(
        _combine_body,
        grid=(n // ROW_BLK,),
        in_specs=[spec_t, spec_t, spec_v, spec_t, spec_t, spec_v],
        out_specs=[spec_t, spec_t, spec_t, spec_t],
        out_shape=[jax.ShapeDtypeStruct((n, D), jnp.float32)] * 4,
    )(acc_u, old_u, d_u, acc_i, old_i, d_i)


def _prescale_body(oldu_ref, du_ref, oldi_ref, di_ref, su_ref, si_ref):
    su_ref[...] = jnp.sqrt(du_ref[...]) * oldu_ref[...]
    si_ref[...] = jnp.sqrt(di_ref[...]) * oldi_ref[...]


def _prescale(old_u, d_u, old_i, d_i):
    n = old_u.shape[0]
    spec_t = pl.BlockSpec((ROW_BLK, D), lambda i: (i, 0))
    spec_v = pl.BlockSpec((ROW_BLK, 1), lambda i: (i, 0))
    return pl.pallas_call(
        _prescale_body,
        grid=(n // ROW_BLK,),
        in_specs=[spec_t, spec_v, spec_t, spec_v],
        out_specs=[spec_t, spec_t],
        out_shape=[jax.ShapeDtypeStruct((n, D), jnp.float32)] * 2,
    )(old_u, d_u, old_i, d_i)


def kernel(emb_user, emb_item, ui_vals, iu_vals, d_users, d_items,
           users, pos_item, neg_item, rows, cols):
    gu = [emb_user]
    gi = [emb_item]
    zrows = jnp.zeros((ZBLK, D), jnp.float32)
    dstU, srcU, dstI, srcI, cnt = _bucket(rows, cols)
    su, si = _prescale(emb_user, d_users, emb_item, d_items)
    for _ in range(LAYERS):
        acc_u, acc_i = _spmm_pair(zrows, si, su, dstU, srcU, dstI, srcI, cnt)
        new_u, su, new_i, si = _combine(acc_u, gu[-1], d_users,
                                        acc_i, gi[-1], d_items)
        gu.append(new_u)
        gi.append(new_i)

    pieces = _final_gather(gu, gi, users, pos_item, neg_item)
    u_emb = jnp.concatenate(pieces[0:4], axis=1)
    pos_emb = jnp.concatenate(pieces[4:8], axis=1)
    neg_emb = jnp.concatenate(pieces[8:12], axis=1)
    return u_emb, pos_emb, neg_emb


# speedup vs baseline: 8.9643x; 1.1111x over previous
import functools

import jax
import jax.numpy as jnp
from jax import lax
from jax.experimental import pallas as pl
from jax.experimental.pallas import tpu as pltpu
from jax.experimental.pallas import tpu_sc as plsc

N_USERS = 50000
N_ITEMS = 50000
NNZ = 800000
D = 64
LAYERS = 3
BATCH = 4096

_SC_PARAMS = pltpu.CompilerParams(needs_layout_passes=False,
                                  use_tc_tiling_on_sc=False)

NC = 2
NS = 16
NW = NC * NS
LANES = 16

HALF = N_USERS // NC
TRASH = HALF
ACC_ROWS = 25200
COPY_BLK = 200
ZBLK = ACC_ROWS // NS
CHUNK = 128
NCHT = NNZ // CHUNK
NCHW = 196
SLOT_CAP = NCHW * CHUNK
SENTINEL = 2 * N_USERS

ROW_BLK = 2000


def _sc_mesh():
    return plsc.VectorSubcoreMesh(core_axis_name="c", subcore_axis_name="s",
                                  num_cores=NC, num_subcores=NS)


def _bucket_body(rows, cols, dstU, srcU, dstI, srcI, cnt,
                 ebufd, ebufs, sd0, ss0, sd1, ss1, cntb,
                 sem0, sem1, sem2, sem3):
    c = lax.axis_index("c")
    s = lax.axis_index("s")
    w = s * NC + c
    nch = 195 + jnp.where(w < 10, 1, 0)
    iota16 = lax.iota(jnp.int32, LANES)
    sems = (sem0, sem1, sem2, sem3)

    def pass_dir(dst_arr, src_arr, outD, outS, dir_idx, cv_in):
        def issue(j, b):
            off = (j * NW + w) * CHUNK
            pltpu.async_copy(dst_arr.at[pl.ds(off, CHUNK)], ebufd.at[b],
                             sems[b])
            pltpu.async_copy(src_arr.at[pl.ds(off, CHUNK)], ebufs.at[b],
                             sems[b])

        def wait(j, b):
            off = (j * NW + w) * CHUNK
            pltpu.make_async_copy(dst_arr.at[pl.ds(off, CHUNK)],
                                  ebufd.at[b], sems[b]).wait()
            pltpu.make_async_copy(src_arr.at[pl.ds(off, CHUNK)],
                                  ebufs.at[b], sems[b]).wait()

        for j0 in (0, 1, 2):
            @pl.when(j0 < nch)
            def _():
                issue(j0, j0)

        def pair(p, carry):
            off0, off1 = carry
            for b in (0, 1, 2, 3):
                j = p * 4 + b
                valid = j < nch

                @pl.when(valid)
                def _():
                    wait(j, b)

                @pl.when(j + 3 < nch)
                def _():
                    issue(j + 3, (b + 3) % 4)

                for q in range(CHUNK // LANES):
                    d = ebufd[b, pl.ds(q * LANES, LANES)]
                    sv = ebufs[b, pl.ds(q * LANES, LANES)]
                    m0 = jnp.logical_and(d < HALF, valid)
                    m1 = jnp.logical_and(d >= HALF, valid)
                    i0 = m0.astype(jnp.int32)
                    cs0 = plsc.cumsum(i0)
                    p0 = off0 + cs0 - 1
                    p1 = off1 + iota16 - cs0
                    plsc.store_scatter(sd0, [p0], d, mask=m0)
                    plsc.store_scatter(ss0, [p0], sv, mask=m0)
                    plsc.store_scatter(sd1, [p1], d, mask=m1)
                    plsc.store_scatter(ss1, [p1], sv, mask=m1)
                    s0 = jnp.sum(i0)
                    off0 = off0 + s0
                    off1 = off1 + jnp.where(valid, LANES - s0, 0)
            return off0, off1

        n0, n1 = lax.fori_loop(0, NCHW // 4, pair,
                               (jnp.int32(0), jnp.int32(0)))

        def finish(n, sdst, ssrc, outDref, outSref):
            pend = jnp.minimum((n + CHUNK - 1) & (-CHUNK), SLOT_CAP)
            sent = jnp.full((LANES,), SENTINEL, jnp.int32)
            zero = jnp.zeros((LANES,), jnp.int32)
            for t in range(CHUNK // LANES):
                pos = n + t * LANES + iota16
                m = pos < pend
                plsc.store_scatter(sdst, [pos], sent, mask=m)
                plsc.store_scatter(ssrc, [pos], zero, mask=m)
            pltpu.sync_copy(sdst, outDref)
            pltpu.sync_copy(ssrc, outSref)
            return pend // CHUNK

        nch0 = finish(n0, sd0, ss0, outD.at[0, w], outS.at[0, w])
        nch1 = finish(n1, sd1, ss1, outD.at[1, w], outS.at[1, w])
        cv = jnp.where(iota16 == 2 * dir_idx, nch0,
                       jnp.where(iota16 == 2 * dir_idx + 1, nch1, cv_in))
        return cv

    cv = jnp.zeros((LANES,), jnp.int32)
    cv = pass_dir(rows, cols, dstU, srcU, 0, cv)
    cv = pass_dir(cols, rows, dstI, srcI, 1, cv)
    cntb[...] = cv
    pltpu.sync_copy(cntb, cnt.at[w])


def _bucket(rows, cols):
    slot = jax.ShapeDtypeStruct((2, NW, SLOT_CAP), jnp.int32)
    f = pl.kernel(
        _bucket_body,
        out_type=[slot, slot, slot, slot,
                  jax.ShapeDtypeStruct((NW, LANES), jnp.int32)],
        mesh=_sc_mesh(),
        scratch_types=[
            pltpu.VMEM((4, CHUNK), jnp.int32),
            pltpu.VMEM((4, CHUNK), jnp.int32),
            pltpu.VMEM((SLOT_CAP,), jnp.int32),
            pltpu.VMEM((SLOT_CAP,), jnp.int32),
            pltpu.VMEM((SLOT_CAP,), jnp.int32),
            pltpu.VMEM((SLOT_CAP,), jnp.int32),
            pltpu.VMEM((LANES,), jnp.int32),
        ] + [pltpu.SemaphoreType.DMA] * 4,
        compiler_params=_SC_PARAMS,
    )
    return f(rows, cols)


def _spmm_body(dir_idx, zrows, src_table, dArr, sArr, cnt, out_hbm,
               dstb, srcb, gbuf, cbuf, loc0, loc1, loc2, acc,
               sem_i0, sem_i1, sem_i2, sem_g0, sem_g1, sem_g2,
               sem_s0, sem_s1, sem_s2):
    c = lax.axis_index("c")
    s = lax.axis_index("s")
    lo = c * HALF
    sem_i = (sem_i0, sem_i1, sem_i2)
    sem_g = (sem_g0, sem_g1, sem_g2)
    sem_s = (sem_s0, sem_s1, sem_s2)

    iota16 = lax.iota(jnp.int32, LANES)

    def get_count(slot, col):
        pltpu.sync_copy(cnt.at[slot], cbuf)
        v = cbuf[...]
        return jnp.sum(jnp.where(iota16 == col, v, 0))

    locs = (loc0, loc1, loc2)

    if True:
        pltpu.sync_copy(zrows, acc.at[pl.ds(s * ZBLK, ZBLK)])
        plsc.subcore_barrier()

        col = 2 * dir_idx + c
        nA = get_count(s, col)
        nB = get_count(s + NS, col)
        T = nA + nB

        def chunk_src(t, arr):
            inA = t < nA
            slot = jnp.where(inA, s, s + NS)
            chv = jnp.where(inA, t, t - nA)
            return arr.at[c, slot, pl.ds(chv * CHUNK, CHUNK)]

        def issue_idx(t, b):
            pltpu.async_copy(chunk_src(t, dArr), dstb.at[b], sem_i[b])
            pltpu.async_copy(chunk_src(t, sArr), srcb.at[b], sem_i[b])

        def wait_idx(t, b):
            pltpu.make_async_copy(chunk_src(t, dArr), dstb.at[b],
                                  sem_i[b]).wait()
            pltpu.make_async_copy(chunk_src(t, sArr), srcb.at[b],
                                  sem_i[b]).wait()

        def issue_gather(b):
            pltpu.async_copy(src_table.at[srcb.at[b]], gbuf.at[b], sem_g[b])

        def wait_gather(b):
            pltpu.make_async_copy(src_table.at[srcb.at[b]], gbuf.at[b],
                                  sem_g[b]).wait()

        def do_scatters(b):
            lref = locs[b]
            for j in range(CHUNK // LANES):
                d = dstb[b, pl.ds(j * LANES, LANES)]
                keep = jnp.logical_and(d >= lo, d < lo + HALF)
                loc = jnp.where(keep, d - lo, TRASH + (d & 63))
                plsc.store_scatter(lref, [j * LANES + iota16], loc)
            pltpu.async_copy(gbuf.at[b], acc.at[lref], sem_s[b], add=True)

        def wait_scatters(b):
            pltpu.make_async_copy(gbuf.at[b], acc.at[locs[b]],
                                  sem_s[b]).wait()

        for t0 in (0, 1, 2):
            @pl.when(t0 < T)
            def _():
                issue_idx(t0, t0)

        for t0 in (0, 1):
            @pl.when(t0 < T)
            def _():
                wait_idx(t0, t0)
                issue_gather(t0)

        @pl.loop(0, (T + 3) // 3)
        def _(p):
            for b in (0, 1, 2):
                t = p * 3 + b

                @pl.when(jnp.logical_and(t >= 1, t - 1 < T))
                def _():
                    wait_scatters((b + 2) % 3)

                @pl.when(t + 2 < T)
                def _():
                    wait_idx(t + 2, (b + 2) % 3)
                    issue_gather((b + 2) % 3)

                @pl.when(t < T)
                def _():
                    wait_gather(b)
                    do_scatters(b)

                @pl.when(t + 3 < T)
                def _():
                    issue_idx(t + 3, b)

        plsc.subcore_barrier()

        @pl.loop(s, HALF // COPY_BLK, step=NS)
        def _(ch):
            pltpu.sync_copy(acc.at[pl.ds(ch * COPY_BLK, COPY_BLK)],
                            out_hbm.at[pl.ds(lo + ch * COPY_BLK, COPY_BLK)])


def _spmm_one(zrows, src_table, dArr, sArr, cnt, dir_idx):
    f = pl.kernel(
        functools.partial(_spmm_body, dir_idx),
        out_type=jax.ShapeDtypeStruct((N_USERS, D), jnp.float32),
        mesh=_sc_mesh(),
        scratch_types=[
            pltpu.VMEM((3, CHUNK), jnp.int32),
            pltpu.VMEM((3, CHUNK), jnp.int32),
            pltpu.VMEM((3, CHUNK, D), jnp.float32),
            pltpu.VMEM((LANES,), jnp.int32),
            pltpu.VMEM((CHUNK,), jnp.int32),
            pltpu.VMEM((CHUNK,), jnp.int32),
            pltpu.VMEM((CHUNK,), jnp.int32),
            pltpu.VMEM_SHARED((ACC_ROWS, D), jnp.float32),
        ] + [pltpu.SemaphoreType.DMA] * 9,
        compiler_params=_SC_PARAMS,
    )
    return f(zrows, src_table, dArr, sArr, cnt)


def _final_gather_body(tabs, idxs, outs, idxb, gb, sem):
    c = lax.axis_index("c")
    s = lax.axis_index("s")
    w = s * NC + c
    n = BATCH // NW
    base = w * n
    for r, idx_arr in enumerate(idxs):
        pltpu.sync_copy(idx_arr.at[pl.ds(base, n)], idxb.at[r])
    for t in range(4 * len(idxs)):
        r = t // 4
        tab = tabs[(0 if r == 0 else 1) * 4 + t % 4]
        pltpu.sync_copy(tab.at[idxb.at[r]], gb)
        pltpu.sync_copy(gb, outs[t].at[pl.ds(base, n)])


def _final_gather(gu, gi, users, pos_item, neg_item):
    def body(u0, u1, u2, u3, i0, i1, i2, i3, users_, pos_, neg_, *rest):
        outs = rest[:12]
        idxb, gb, sem = rest[12:]
        _final_gather_body((u0, u1, u2, u3, i0, i1, i2, i3),
                           (users_, pos_, neg_), outs, idxb, gb, sem)

    f = pl.kernel(
        body,
        out_type=[jax.ShapeDtypeStruct((BATCH, D), jnp.float32)] * 12,
        mesh=_sc_mesh(),
        scratch_types=[
            pltpu.VMEM((3, BATCH // NW), jnp.int32),
            pltpu.VMEM((BATCH // NW, D), jnp.float32),
            pltpu.SemaphoreType.DMA,
        ],
        compiler_params=_SC_PARAMS,
    )
    return f(*gu, *gi, users, pos_item, neg_item)


def _combine_body(acc_ref, old_ref, d_ref, new_ref, s_ref):
    a = jnp.sqrt(d_ref[...])
    new = a * acc_ref[...] + d_ref[...] * old_ref[...]
    new_ref[...] = new
    s_ref[...] = a * new


def _combine(acc, old, dvec):
    n = acc.shape[0]
    spec_t = pl.BlockSpec((ROW_BLK, D), lambda i: (i, 0))
    spec_v = pl.BlockSpec((ROW_BLK, 1), lambda i: (i, 0))
    return pl.pallas_call(
        _combine_body,
        grid=(n // ROW_BLK,),
        in_specs=[spec_t, spec_t, spec_v],
        out_specs=[spec_t, spec_t],
        out_shape=[jax.ShapeDtypeStruct((n, D), jnp.float32)] * 2,
    )(acc, old, dvec)


def _prescale_body(oldu_ref, du_ref, oldi_ref, di_ref, su_ref, si_ref):
    su_ref[...] = jnp.sqrt(du_ref[...]) * oldu_ref[...]
    si_ref[...] = jnp.sqrt(di_ref[...]) * oldi_ref[...]


def _prescale(old_u, d_u, old_i, d_i):
    n = old_u.shape[0]
    spec_t = pl.BlockSpec((ROW_BLK, D), lambda i: (i, 0))
    spec_v = pl.BlockSpec((ROW_BLK, 1), lambda i: (i, 0))
    return pl.pallas_call(
        _prescale_body,
        grid=(n // ROW_BLK,),
        in_specs=[spec_t, spec_v, spec_t, spec_v],
        out_specs=[spec_t, spec_t],
        out_shape=[jax.ShapeDtypeStruct((n, D), jnp.float32)] * 2,
    )(old_u, d_u, old_i, d_i)


def kernel(emb_user, emb_item, ui_vals, iu_vals, d_users, d_items,
           users, pos_item, neg_item, rows, cols):
    gu = [emb_user]
    gi = [emb_item]
    zrows = jnp.zeros((ZBLK, D), jnp.float32)
    dstU, srcU, dstI, srcI, cnt = _bucket(rows, cols)
    su, si = _prescale(emb_user, d_users, emb_item, d_items)
    for _ in range(LAYERS):
        acc_u = _spmm_one(zrows, si, dstU, srcU, cnt, 0)
        new_u, su_new = _combine(acc_u, gu[-1], d_users)
        acc_i = _spmm_one(zrows, su, dstI, srcI, cnt, 1)
        new_i, si = _combine(acc_i, gi[-1], d_items)
        su = su_new
        gu.append(new_u)
        gi.append(new_i)

    pieces = _final_gather(gu, gi, users, pos_item, neg_item)
    u_emb = jnp.concatenate(pieces[0:4], axis=1)
    pos_emb = jnp.concatenate(pieces[4:8], axis=1)
    neg_emb = jnp.concatenate(pieces[8:12], axis=1)
    return u_emb, pos_emb, neg_emb
